# Initial kernel scaffold; baseline (speedup 1.0000x reference)
import functools

import jax
import jax.numpy as jnp
from jax import lax
from jax.experimental import pallas as pl
from jax.experimental.pallas import tpu as pltpu
from jax.experimental.pallas import tpu_sc as plsc

_N = 10000
_NP = 10240
_E = 320000
_NC, _NS = 2, 16
_NW = _NC * _NS
_EPW = 10240
_EP = _NW * _EPW
_C = 512
_NCH = _EPW // _C
_IW = 128
_NJ = _C // _IW
_F32 = jnp.float32


def _dyn_gather(v, idx):
    dnums = lax.GatherDimensionNumbers(
        offset_dims=(), collapsed_slice_dims=(0,), start_index_map=(0,))
    return lax.gather(v, idx[:, None], dnums, (1,),
                      mode=lax.GatherScatterMode.PROMISE_IN_BOUNDS)


def _tc1(xp, qk0, qb0, kk0, kb0, k0, p8):
    def body(x_ref, qk_ref, qb_ref, kk_ref, kb_ref, kv_ref, p_ref,
             qkt_ref, v_ref, es_ref, outs_ref):
        x = x_ref[...]
        q = jnp.maximum(jnp.dot(x, qk_ref[...], preferred_element_type=_F32)
                        + qb_ref[...], 0.0)
        k = jnp.maximum(jnp.dot(x, kk_ref[...], preferred_element_type=_F32)
                        + kb_ref[...], 0.0)
        v = jnp.dot(x, kv_ref[...], preferred_element_type=_F32)
        es = jnp.exp(q * k)
        es64 = jnp.dot(es, p_ref[...], preferred_element_type=_F32)
        qkt_ref[...] = jnp.concatenate([q, k], axis=1)
        v_ref[...] = v
        es_ref[...] = es
        outs_ref[...] = es64 * v
    return pl.pallas_call(
        body,
        out_shape=(
            jax.ShapeDtypeStruct((_NP, 16), _F32),
            jax.ShapeDtypeStruct((_NP, 64), _F32),
            jax.ShapeDtypeStruct((_NP, 8), _F32),
            jax.ShapeDtypeStruct((_NP, 64), _F32),
        ),
    )(xp, qk0, qb0, kk0, kb0, k0, p8)


def _tc2(accp, es0, outs0, p8, b0, qk1, qb1, kk1, kb1, k1p):
    def body(a_ref, es_ref, os_ref, p_ref, b0_ref,
             qk_ref, qb_ref, kk_ref, kb_ref, kv_ref,
             q1_ref, k1_ref, v1_ref, es1_ref, outs1_ref):
        acc = a_ref[0:_NP, :] + a_ref[_NP:2 * _NP, :]
        s = acc[:, 64:72] + es_ref[...]
        s64 = jnp.dot(s, p_ref[...], preferred_element_type=_F32)
        h = jnp.maximum((acc[:, 0:64] + os_ref[...]) / s64 + b0_ref[...], 0.0)
        q1 = jnp.maximum(jnp.dot(h, qk_ref[...], preferred_element_type=_F32)
                         + qb_ref[...], 0.0)
        k1 = jnp.maximum(jnp.dot(h, kk_ref[...], preferred_element_type=_F32)
                         + kb_ref[...], 0.0)
        v1 = jnp.dot(h, kv_ref[...], preferred_element_type=_F32)
        es1 = jnp.exp(q1 * k1)
        q1_ref[...] = q1
        k1_ref[...] = k1
        v1_ref[...] = v1
        es1_ref[...] = es1
        outs1_ref[...] = es1 * v1
    return pl.pallas_call(
        body,
        out_shape=(
            jax.ShapeDtypeStruct((_NP, 1), _F32),
            jax.ShapeDtypeStruct((_NP, 1), _F32),
            jax.ShapeDtypeStruct((_NP, 48), _F32),
            jax.ShapeDtypeStruct((_NP, 1), _F32),
            jax.ShapeDtypeStruct((_NP, 48), _F32),
        ),
    )(accp, es0, outs0, p8, b0, qk1, qb1, kk1, kb1, k1p)


def _tc3(acc1p, es1, outs1, b1p):
    def body(a_ref, es_ref, os_ref, b_ref, o_ref):
        acc = a_ref[0:_NP, :] + a_ref[_NP:2 * _NP, :]
        s1 = acc[:, 48:49] + es_ref[...]
        o_ref[...] = (acc[:, 0:48] + os_ref[...]) / s1 + b_ref[...]
    return pl.pallas_call(
        body,
        out_shape=jax.ShapeDtypeStruct((_NP, 48), _F32),
    )(acc1p, es1, outs1, b1p)


def _mesh():
    return plsc.VectorSubcoreMesh(core_axis_name="c", subcore_axis_name="s",
                                  num_cores=_NC, num_subcores=_NS)


def _scl0(rowp, colp, qkt, v0):
    @functools.partial(
        pl.kernel,
        out_type=jax.ShapeDtypeStruct((2 * _NP, 80), _F32),
        mesh=_mesh(),
        scratch_types=[
            pltpu.VMEM_SHARED((_NP, 80), _F32),
            pltpu.VMEM((_NJ, _IW), jnp.int32),
            pltpu.VMEM((_NJ, _IW), jnp.int32),
            pltpu.VMEM((_C, 16), _F32),
            pltpu.VMEM((_C, 16), _F32),
            pltpu.VMEM((_C, 64), _F32),
            pltpu.VMEM((_C, 80), _F32),
            pltpu.SemaphoreType.DMA,
        ],
    )
    def k(row_hbm, col_hbm, qkt_hbm, v_hbm, out_hbm,
          acc_sh, ridx, cidx, qa, kb, vb, wb, sem):
        cid = lax.axis_index("c")
        sid = lax.axis_index("s")
        wid = cid * _NS + sid
        lane = lax.iota(jnp.int32, 16)
        rot8 = lane ^ 8
        hsel = lane >> 3
        zf = (lane * 0).astype(_F32)

        @pl.loop(0, _C)
        def _(i):
            for k2 in range(5):
                wb[i, pl.ds(16 * k2, 16)] = zf
        pltpu.sync_copy(wb, acc_sh.at[pl.ds(sid * 640, 512)])
        pltpu.sync_copy(wb.at[pl.ds(0, 128)],
                        acc_sh.at[pl.ds(sid * 640 + 512, 128)])
        plsc.subcore_barrier()

        @pl.loop(0, _NCH)
        def _(ch):
            r0 = (wid * _EPW + ch * _C) // _IW
            d1 = pltpu.async_copy(row_hbm.at[pl.ds(r0, _NJ)], ridx, sem)
            d2 = pltpu.async_copy(col_hbm.at[pl.ds(r0, _NJ)], cidx, sem)
            d1.wait()
            d2.wait()
            gd = []
            for j in range(_NJ):
                dst = pl.ds(j * _IW, _IW)
                gd.append(pltpu.async_copy(qkt_hbm.at[ridx.at[j]],
                                           qa.at[dst], sem))
                gd.append(pltpu.async_copy(qkt_hbm.at[cidx.at[j]],
                                           kb.at[dst], sem))
                gd.append(pltpu.async_copy(v_hbm.at[cidx.at[j]],
                                           vb.at[dst], sem))
            for d in gd:
                d.wait()

            @pl.loop(0, _C)
            def _(i):
                a = qa[i, :]
                b = kb[i, :]
                e = jnp.exp(a * _dyn_gather(b, rot8))
                for k2 in range(4):
                    ev = _dyn_gather(e, hsel + 2 * k2)
                    wb[i, pl.ds(16 * k2, 16)] = vb[i, pl.ds(16 * k2, 16)] * ev
                wb[i, pl.ds(64, 16)] = jnp.where(lane < 8, e, 0.0)

            sd = []
            for j in range(_NJ):
                sd.append(pltpu.async_copy(wb.at[pl.ds(j * _IW, _IW)],
                                           acc_sh.at[ridx.at[j]], sem,
                                           add=True))
            for d in sd:
                d.wait()

        plsc.subcore_barrier()
        pltpu.sync_copy(acc_sh.at[pl.ds(sid * 640, 640)],
                        out_hbm.at[pl.ds(cid * _NP + sid * 640, 640)])

    return k(rowp, colp, qkt, v0)


def _scl1(rowp, colp, rowf, colf, q1t, k1t, v1):
    @functools.partial(
        pl.kernel,
        out_type=jax.ShapeDtypeStruct((2 * _NP, 64), _F32),
        mesh=_mesh(),
        scratch_types=[
            pltpu.VMEM_SHARED((_NP, 64), _F32),
            pltpu.VMEM((_NJ, _IW), jnp.int32),
            pltpu.VMEM((_NJ, _IW), jnp.int32),
            pltpu.VMEM((_C,), jnp.int32),
            pltpu.VMEM((_C,), jnp.int32),
            pltpu.VMEM((_NP,), _F32),
            pltpu.VMEM((_NP,), _F32),
            pltpu.VMEM((_C, 48), _F32),
            pltpu.VMEM((_C, 64), _F32),
            pltpu.SemaphoreType.DMA,
        ],
    )
    def k(row_hbm, col_hbm, rowf_hbm, colf_hbm, q1_hbm, k1_hbm, v1_hbm,
          out_hbm, acc_sh, ridx, cidx, ridxf, cidxf, q1t, k1t, vb, wb, sem):
        cid = lax.axis_index("c")
        sid = lax.axis_index("s")
        wid = cid * _NS + sid
        lane = lax.iota(jnp.int32, 16)
        zf = (lane * 0).astype(_F32)

        pltpu.sync_copy(q1_hbm, q1t)
        pltpu.sync_copy(k1_hbm, k1t)

        @pl.loop(0, _C)
        def _(i):
            for k2 in range(4):
                wb[i, pl.ds(16 * k2, 16)] = zf
        pltpu.sync_copy(wb, acc_sh.at[pl.ds(sid * 640, 512)])
        pltpu.sync_copy(wb.at[pl.ds(0, 128)],
                        acc_sh.at[pl.ds(sid * 640 + 512, 128)])
        plsc.subcore_barrier()

        @pl.loop(0, _NCH)
        def _(ch):
            base = wid * _EPW + ch * _C
            r0 = base // _IW
            d1 = pltpu.async_copy(row_hbm.at[pl.ds(r0, _NJ)], ridx, sem)
            d2 = pltpu.async_copy(col_hbm.at[pl.ds(r0, _NJ)], cidx, sem)
            d3 = pltpu.async_copy(rowf_hbm.at[pl.ds(base, _C)], ridxf, sem)
            d4 = pltpu.async_copy(colf_hbm.at[pl.ds(base, _C)], cidxf, sem)
            d1.wait()
            d2.wait()
            d3.wait()
            d4.wait()
            gd = []
            for j in range(_NJ):
                gd.append(pltpu.async_copy(v1_hbm.at[cidx.at[j]],
                                           vb.at[pl.ds(j * _IW, _IW)], sem))
            for d in gd:
                d.wait()

            @pl.loop(0, _C // 16)
            def _(g):
                idxr = ridxf[pl.ds(g * 16, 16)]
                idxc = cidxf[pl.ds(g * 16, 16)]
                qg = plsc.load_gather(q1t, [idxr])
                kg = plsc.load_gather(k1t, [idxc])
                e1 = jnp.exp(qg * kg)
                for i in range(16):
                    sp = _dyn_gather(e1, lane * 0 + i)
                    ei = g * 16 + i
                    for k2 in range(3):
                        wb[ei, pl.ds(16 * k2, 16)] = (
                            vb[ei, pl.ds(16 * k2, 16)] * sp)
                    wb[ei, pl.ds(48, 16)] = jnp.where(lane < 1, sp, 0.0)

            sd = []
            for j in range(_NJ):
                sd.append(pltpu.async_copy(wb.at[pl.ds(j * _IW, _IW)],
                                           acc_sh.at[ridx.at[j]], sem,
                                           add=True))
            for d in sd:
                d.wait()

        plsc.subcore_barrier()
        pltpu.sync_copy(acc_sh.at[pl.ds(sid * 640, 640)],
                        out_hbm.at[pl.ds(cid * _NP + sid * 640, 640)])

    return k(rowp, colp, rowf, colf, q1t, k1t, v1)


def kernel(x, edge_index, edge_weight, qk0, qb0, kk0, kb0, k0, b0,
           qk1, qb1, kk1, kb1, k1, b1):
    del edge_weight
    xp = jnp.zeros((_NP, 128), _F32).at[:_N].set(x)
    row = edge_index[0].astype(jnp.int32)
    col = edge_index[1].astype(jnp.int32)
    pad = jnp.full((_EP - _E,), _N, jnp.int32)
    rowf = jnp.concatenate([row, pad])
    colf = jnp.concatenate([col, pad])
    rowp = rowf.reshape(_EP // _IW, _IW)
    colp = colf.reshape(_EP // _IW, _IW)
    p8 = jnp.kron(jnp.eye(8, dtype=_F32), jnp.ones((1, 8), _F32))

    qkt, v0, es0, outs0 = _tc1(xp, qk0, qb0.reshape(1, 8), kk0,
                               kb0.reshape(1, 8), k0, p8)
    accp = _scl0(rowp, colp, qkt, v0)
    k1p = jnp.zeros((64, 48), _F32).at[:, :40].set(k1)
    q1, k1c, v1, es1, outs1 = _tc2(accp, es0, outs0, p8, b0.reshape(1, 64),
                                   qk1, qb1.reshape(1, 1), kk1,
                                   kb1.reshape(1, 1), k1p)
    acc1p = _scl1(rowp, colp, rowf, colf, q1.reshape(_NP), k1c.reshape(_NP),
                  v1)
    b1p = jnp.zeros((1, 48), _F32).at[0, :40].set(b1)
    out = _tc3(acc1p, es1, outs1, b1p)
    return out[:_N, :40]


# speedup vs baseline: 47.4973x; 47.4973x over previous
import functools

import jax
import jax.numpy as jnp
from jax import lax
from jax.experimental import pallas as pl
from jax.experimental.pallas import tpu as pltpu
from jax.experimental.pallas import tpu_sc as plsc

_N = 10000
_NP = 10240
_E = 320000
_NC, _NS = 2, 16
_NW = _NC * _NS
_EPW = 10240
_EP = _NW * _EPW
_C = 256
_SC = 1024
_NSC = _EPW // _SC
_IW = 128
_NJ = _C // _IW
_F32 = jnp.float32


def _dyn_gather(v, idx):
    dnums = lax.GatherDimensionNumbers(
        offset_dims=(), collapsed_slice_dims=(0,), start_index_map=(0,))
    return lax.gather(v, idx[:, None], dnums, (1,),
                      mode=lax.GatherScatterMode.PROMISE_IN_BOUNDS)


def _tc1(xp, qk0, qb0, kk0, kb0, k0, p8):
    def body(x_ref, qk_ref, qb_ref, kk_ref, kb_ref, kv_ref, p_ref,
             qkt_ref, v_ref, es_ref, outs_ref):
        x = x_ref[...]
        q = jnp.maximum(jnp.dot(x, qk_ref[...], preferred_element_type=_F32)
                        + qb_ref[...], 0.0)
        k = jnp.maximum(jnp.dot(x, kk_ref[...], preferred_element_type=_F32)
                        + kb_ref[...], 0.0)
        v = jnp.dot(x, kv_ref[...], preferred_element_type=_F32)
        es = jnp.exp(q * k)
        es64 = jnp.dot(es, p_ref[...], preferred_element_type=_F32)
        qkt_ref[...] = jnp.concatenate([q, k], axis=1)
        v_ref[...] = v
        es_ref[...] = es
        outs_ref[...] = es64 * v
    return pl.pallas_call(
        body,
        out_shape=(
            jax.ShapeDtypeStruct((_NP, 16), _F32),
            jax.ShapeDtypeStruct((_NP, 64), _F32),
            jax.ShapeDtypeStruct((_NP, 8), _F32),
            jax.ShapeDtypeStruct((_NP, 64), _F32),
        ),
    )(xp, qk0, qb0, kk0, kb0, k0, p8)


def _tc2(accp, es0, outs0, p8, b0, qk1, qb1, kk1, kb1, k1p):
    def body(a_ref, es_ref, os_ref, p_ref, b0_ref,
             qk_ref, qb_ref, kk_ref, kb_ref, kv_ref,
             q1_ref, k1_ref, v1_ref, es1_ref, outs1_ref):
        acc = a_ref[0:_NP, :] + a_ref[_NP:2 * _NP, :]
        s = acc[:, 64:72] + es_ref[...]
        s64 = jnp.dot(s, p_ref[...], preferred_element_type=_F32)
        h = jnp.maximum((acc[:, 0:64] + os_ref[...]) / s64 + b0_ref[...], 0.0)
        q1 = jnp.maximum(jnp.dot(h, qk_ref[...], preferred_element_type=_F32)
                         + qb_ref[...], 0.0)
        k1 = jnp.maximum(jnp.dot(h, kk_ref[...], preferred_element_type=_F32)
                         + kb_ref[...], 0.0)
        v1 = jnp.dot(h, kv_ref[...], preferred_element_type=_F32)
        es1 = jnp.exp(q1 * k1)
        q1_ref[...] = q1
        k1_ref[...] = k1
        v1_ref[...] = v1
        es1_ref[...] = es1
        outs1_ref[...] = es1 * v1
    return pl.pallas_call(
        body,
        out_shape=(
            jax.ShapeDtypeStruct((_NP, 1), _F32),
            jax.ShapeDtypeStruct((_NP, 1), _F32),
            jax.ShapeDtypeStruct((_NP, 48), _F32),
            jax.ShapeDtypeStruct((_NP, 1), _F32),
            jax.ShapeDtypeStruct((_NP, 48), _F32),
        ),
    )(accp, es0, outs0, p8, b0, qk1, qb1, kk1, kb1, k1p)


def _tc3(acc1p, es1, outs1, b1p):
    def body(a_ref, es_ref, os_ref, b_ref, o_ref):
        acc = a_ref[0:_NP, :] + a_ref[_NP:2 * _NP, :]
        s1 = acc[:, 48:49] + es_ref[...]
        o_ref[...] = (acc[:, 0:48] + os_ref[...]) / s1 + b_ref[...]
    return pl.pallas_call(
        body,
        out_shape=jax.ShapeDtypeStruct((_NP, 48), _F32),
    )(acc1p, es1, outs1, b1p)


def _mesh():
    return plsc.VectorSubcoreMesh(core_axis_name="c", subcore_axis_name="s",
                                  num_cores=_NC, num_subcores=_NS)


_SC_PARAMS = pltpu.CompilerParams(use_tc_tiling_on_sc=False,
                                  needs_layout_passes=False)


def _scl0(rowp, colp, qkt, v0):
    @functools.partial(
        pl.kernel,
        out_type=jax.ShapeDtypeStruct((2 * _NP, 80), _F32),
        mesh=_mesh(),
        compiler_params=_SC_PARAMS,
        scratch_types=[
            pltpu.VMEM_SHARED((_NP, 80), _F32),
            pltpu.VMEM((8, _IW), jnp.int32),
            pltpu.VMEM((8, _IW), jnp.int32),
            pltpu.VMEM((_C, 16), _F32),
            pltpu.VMEM((_C, 16), _F32),
            pltpu.VMEM((_C, 64), _F32),
            pltpu.VMEM((_C, 80), _F32),
            pltpu.SemaphoreType.DMA,
        ],
    )
    def k(row_hbm, col_hbm, qkt_hbm, v_hbm, out_hbm,
          acc_sh, ridx, cidx, qa, kb, vb, wb, sem):
        cid = lax.axis_index("c")
        sid = lax.axis_index("s")
        wid = cid * _NS + sid
        lane = lax.iota(jnp.int32, 16)
        rot8 = lane ^ 8
        hsel = lane >> 3
        zf = (lane * 0).astype(_F32)

        @pl.loop(0, _C)
        def _(i):
            for k2 in range(5):
                wb[i, pl.ds(16 * k2, 16)] = zf
        pltpu.sync_copy(wb, acc_sh.at[pl.ds(sid * 640, _C)])
        pltpu.sync_copy(wb, acc_sh.at[pl.ds(sid * 640 + _C, _C)])
        pltpu.sync_copy(wb.at[pl.ds(0, 128)],
                        acc_sh.at[pl.ds(sid * 640 + 2 * _C, 128)])
        plsc.subcore_barrier()

        @pl.loop(0, _NSC)
        def _(sc):
            r0 = pl.multiple_of((wid * _EPW + sc * _SC) // _IW, 8)
            d1 = pltpu.async_copy(row_hbm.at[pl.ds(r0, 8)], ridx, sem)
            d2 = pltpu.async_copy(col_hbm.at[pl.ds(r0, 8)], cidx, sem)
            d1.wait()
            d2.wait()
            for h2 in range(_SC // _C):
                gd = []
                for j in range(_NJ):
                    j8 = h2 * _NJ + j
                    dst = pl.ds(j * _IW, _IW)
                    gd.append(pltpu.async_copy(qkt_hbm.at[ridx.at[j8]],
                                               qa.at[dst], sem))
                    gd.append(pltpu.async_copy(qkt_hbm.at[cidx.at[j8]],
                                               kb.at[dst], sem))
                    gd.append(pltpu.async_copy(v_hbm.at[cidx.at[j8]],
                                               vb.at[dst], sem))
                for d in gd:
                    d.wait()

                @pl.loop(0, _C)
                def _(i):
                    a = qa[i, :]
                    b = kb[i, :]
                    e = jnp.exp(a * _dyn_gather(b, rot8))
                    for k2 in range(4):
                        ev = _dyn_gather(e, hsel + 2 * k2)
                        wb[i, pl.ds(16 * k2, 16)] = (
                            vb[i, pl.ds(16 * k2, 16)] * ev)
                    wb[i, pl.ds(64, 16)] = jnp.where(lane < 8, e, 0.0)

                sd = []
                for j in range(_NJ):
                    sd.append(pltpu.async_copy(
                        wb.at[pl.ds(j * _IW, _IW)],
                        acc_sh.at[ridx.at[h2 * _NJ + j]], sem, add=True))
                for d in sd:
                    d.wait()

        plsc.subcore_barrier()
        pltpu.sync_copy(acc_sh.at[pl.ds(sid * 640, 640)],
                        out_hbm.at[pl.ds(cid * _NP + sid * 640, 640)])

    return k(rowp, colp, qkt, v0)


def _scl1(rowp, colp, q1t, k1t, v1):
    @functools.partial(
        pl.kernel,
        out_type=jax.ShapeDtypeStruct((2 * _NP, 64), _F32),
        mesh=_mesh(),
        compiler_params=_SC_PARAMS,
        scratch_types=[
            pltpu.VMEM_SHARED((_NP, 64), _F32),
            pltpu.VMEM((8, _IW), jnp.int32),
            pltpu.VMEM((8, _IW), jnp.int32),
            pltpu.VMEM((_NP,), _F32),
            pltpu.VMEM((_NP,), _F32),
            pltpu.VMEM((_C, 48), _F32),
            pltpu.VMEM((_C, 64), _F32),
            pltpu.SemaphoreType.DMA,
        ],
    )
    def k(row_hbm, col_hbm, q1_hbm, k1_hbm, v1_hbm,
          out_hbm, acc_sh, ridx, cidx, q1t, k1t, vb, wb, sem):
        cid = lax.axis_index("c")
        sid = lax.axis_index("s")
        wid = cid * _NS + sid
        lane = lax.iota(jnp.int32, 16)
        zf = (lane * 0).astype(_F32)

        pltpu.sync_copy(q1_hbm, q1t)
        pltpu.sync_copy(k1_hbm, k1t)

        @pl.loop(0, _C)
        def _(i):
            for k2 in range(4):
                wb[i, pl.ds(16 * k2, 16)] = zf
        pltpu.sync_copy(wb, acc_sh.at[pl.ds(sid * 640, _C)])
        pltpu.sync_copy(wb, acc_sh.at[pl.ds(sid * 640 + _C, _C)])
        pltpu.sync_copy(wb.at[pl.ds(0, 128)],
                        acc_sh.at[pl.ds(sid * 640 + 2 * _C, 128)])
        plsc.subcore_barrier()

        @pl.loop(0, _NSC)
        def _(sc):
            base = wid * _EPW + sc * _SC
            r0 = pl.multiple_of(base // _IW, 8)
            d1 = pltpu.async_copy(row_hbm.at[pl.ds(r0, 8)], ridx, sem)
            d2 = pltpu.async_copy(col_hbm.at[pl.ds(r0, 8)], cidx, sem)
            d1.wait()
            d2.wait()
            for h2 in range(_SC // _C):
                gd = []
                for j in range(_NJ):
                    gd.append(pltpu.async_copy(
                        v1_hbm.at[cidx.at[h2 * _NJ + j]],
                        vb.at[pl.ds(j * _IW, _IW)], sem))
                for d in gd:
                    d.wait()

                @pl.loop(0, _C // 16)
                def _(g):
                    jrow = h2 * (_C // _IW) + g // 8
                    joff = (g % 8) * 16
                    idxr = ridx[jrow, pl.ds(joff, 16)]
                    idxc = cidx[jrow, pl.ds(joff, 16)]
                    qg = plsc.load_gather(q1t, [idxr])
                    kg = plsc.load_gather(k1t, [idxc])
                    e1 = jnp.exp(qg * kg)
                    for i in range(16):
                        sp = _dyn_gather(e1, lane * 0 + i)
                        ei = g * 16 + i
                        for k2 in range(3):
                            wb[ei, pl.ds(16 * k2, 16)] = (
                                vb[ei, pl.ds(16 * k2, 16)] * sp)
                        wb[ei, pl.ds(48, 16)] = jnp.where(lane < 1, sp, 0.0)

                sd = []
                for j in range(_NJ):
                    sd.append(pltpu.async_copy(
                        wb.at[pl.ds(j * _IW, _IW)],
                        acc_sh.at[ridx.at[h2 * _NJ + j]], sem, add=True))
                for d in sd:
                    d.wait()

        plsc.subcore_barrier()
        pltpu.sync_copy(acc_sh.at[pl.ds(sid * 640, 640)],
                        out_hbm.at[pl.ds(cid * _NP + sid * 640, 640)])

    return k(rowp, colp, q1t, k1t, v1)


def kernel(x, edge_index, edge_weight, qk0, qb0, kk0, kb0, k0, b0,
           qk1, qb1, kk1, kb1, k1, b1):
    del edge_weight
    xp = jnp.zeros((_NP, 128), _F32).at[:_N].set(x)
    row = edge_index[0].astype(jnp.int32)
    col = edge_index[1].astype(jnp.int32)
    pad = jnp.full((_EP - _E,), _N, jnp.int32)
    rowf = jnp.concatenate([row, pad])
    colf = jnp.concatenate([col, pad])
    rowp = rowf.reshape(_EP // _IW, _IW)
    colp = colf.reshape(_EP // _IW, _IW)
    p8 = jnp.kron(jnp.eye(8, dtype=_F32), jnp.ones((1, 8), _F32))

    qkt, v0, es0, outs0 = _tc1(xp, qk0, qb0.reshape(1, 8), kk0,
                               kb0.reshape(1, 8), k0, p8)
    accp = _scl0(rowp, colp, qkt, v0)
    k1p = jnp.zeros((64, 48), _F32).at[:, :40].set(k1)
    q1, k1c, v1, es1, outs1 = _tc2(accp, es0, outs0, p8, b0.reshape(1, 64),
                                   qk1, qb1.reshape(1, 1), kk1,
                                   kb1.reshape(1, 1), k1p)
    acc1p = _scl1(rowp, colp, q1.reshape(_NP), k1c.reshape(_NP), v1)
    b1p = jnp.zeros((1, 48), _F32).at[0, :40].set(b1)
    out = _tc3(acc1p, es1, outs1, b1p)
    return out[:_N, :40]


# speedup vs baseline: 79.2153x; 1.6678x over previous
import functools

import jax
import jax.numpy as jnp
from jax import lax
from jax.experimental import pallas as pl
from jax.experimental.pallas import tpu as pltpu
from jax.experimental.pallas import tpu_sc as plsc

_N = 10000
_NP = 10240
_E = 320000
_NC, _NS = 2, 16
_NW = _NC * _NS
_EPW = 10240
_EP = _NW * _EPW
_CH = 128
_NCH = _EPW // _CH
_F32 = jnp.float32


def _dyn_gather(v, idx):
    dnums = lax.GatherDimensionNumbers(
        offset_dims=(), collapsed_slice_dims=(0,), start_index_map=(0,))
    return lax.gather(v, idx[:, None], dnums, (1,),
                      mode=lax.GatherScatterMode.PROMISE_IN_BOUNDS)


def _tc1(xp, qk0, qb0, kk0, kb0, k0, p8):
    def body(x_ref, qk_ref, qb_ref, kk_ref, kb_ref, kv_ref, p_ref,
             qkt_ref, v_ref, es_ref, outs_ref):
        x = x_ref[...]
        q = jnp.maximum(jnp.dot(x, qk_ref[...], preferred_element_type=_F32)
                        + qb_ref[...], 0.0)
        k = jnp.maximum(jnp.dot(x, kk_ref[...], preferred_element_type=_F32)
                        + kb_ref[...], 0.0)
        v = jnp.dot(x, kv_ref[...], preferred_element_type=_F32)
        es = jnp.exp(q * k)
        es64 = jnp.dot(es, p_ref[...], preferred_element_type=_F32)
        qkt_ref[...] = jnp.concatenate([q, k], axis=1)
        v_ref[...] = v
        es_ref[...] = es
        outs_ref[...] = es64 * v
    return pl.pallas_call(
        body,
        out_shape=(
            jax.ShapeDtypeStruct((_NP, 16), _F32),
            jax.ShapeDtypeStruct((_NP, 64), _F32),
            jax.ShapeDtypeStruct((_NP, 8), _F32),
            jax.ShapeDtypeStruct((_NP, 64), _F32),
        ),
    )(xp, qk0, qb0, kk0, kb0, k0, p8)


def _tc2(accp, es0, outs0, p8, b0, qk1, qb1, kk1, kb1, k1p):
    def body(a_ref, es_ref, os_ref, p_ref, b0_ref,
             qk_ref, qb_ref, kk_ref, kb_ref, kv_ref,
             q1_ref, k1_ref, v1_ref, es1_ref, outs1_ref):
        acc = a_ref[0:_NP, :] + a_ref[_NP:2 * _NP, :]
        s = acc[:, 64:72] + es_ref[...]
        s64 = jnp.dot(s, p_ref[...], preferred_element_type=_F32)
        h = jnp.maximum((acc[:, 0:64] + os_ref[...]) / s64 + b0_ref[...], 0.0)
        q1 = jnp.maximum(jnp.dot(h, qk_ref[...], preferred_element_type=_F32)
                         + qb_ref[...], 0.0)
        k1 = jnp.maximum(jnp.dot(h, kk_ref[...], preferred_element_type=_F32)
                         + kb_ref[...], 0.0)
        v1 = jnp.dot(h, kv_ref[...], preferred_element_type=_F32)
        es1 = jnp.exp(q1 * k1)
        q1_ref[...] = q1
        k1_ref[...] = k1
        v1_ref[...] = v1
        es1_ref[...] = es1
        outs1_ref[...] = es1 * v1
    return pl.pallas_call(
        body,
        out_shape=(
            jax.ShapeDtypeStruct((_NP, 1), _F32),
            jax.ShapeDtypeStruct((_NP, 1), _F32),
            jax.ShapeDtypeStruct((_NP, 48), _F32),
            jax.ShapeDtypeStruct((_NP, 1), _F32),
            jax.ShapeDtypeStruct((_NP, 48), _F32),
        ),
    )(accp, es0, outs0, p8, b0, qk1, qb1, kk1, kb1, k1p)


def _tc3(acc1p, es1, outs1, b1p):
    def body(a_ref, es_ref, os_ref, b_ref, o_ref):
        acc = a_ref[0:_NP, :] + a_ref[_NP:2 * _NP, :]
        s1 = acc[:, 48:49] + es_ref[...]
        o_ref[...] = (acc[:, 0:48] + os_ref[...]) / s1 + b_ref[...]
    return pl.pallas_call(
        body,
        out_shape=jax.ShapeDtypeStruct((_NP, 48), _F32),
    )(acc1p, es1, outs1, b1p)


def _mesh():
    return plsc.VectorSubcoreMesh(core_axis_name="c", subcore_axis_name="s",
                                  num_cores=_NC, num_subcores=_NS)


_SC_PARAMS = pltpu.CompilerParams(use_tc_tiling_on_sc=False,
                                  needs_layout_passes=False)


def _scl0(rowp, colp, qkt, v0):
    @functools.partial(
        pl.kernel,
        out_type=jax.ShapeDtypeStruct((2 * _NP, 80), _F32),
        mesh=_mesh(),
        compiler_params=_SC_PARAMS,
        scratch_types=[
            pltpu.VMEM_SHARED((_NP, 80), _F32),
            pltpu.VMEM((_NCH, _CH), jnp.int32),
            pltpu.VMEM((_NCH, _CH), jnp.int32),
            pltpu.VMEM((_CH, 16), _F32),
            pltpu.VMEM((_CH, 16), _F32),
            pltpu.VMEM((_CH, 16), _F32),
            pltpu.VMEM((_CH, 16), _F32),
            pltpu.VMEM((_CH, 64), _F32),
            pltpu.VMEM((_CH, 64), _F32),
            pltpu.VMEM((_CH, 80), _F32),
            pltpu.VMEM((_CH, 80), _F32),
            pltpu.SemaphoreType.DMA,
            pltpu.SemaphoreType.DMA,
            pltpu.SemaphoreType.DMA,
            pltpu.SemaphoreType.DMA,
            pltpu.SemaphoreType.DMA,
        ],
    )
    def k(row_hbm, col_hbm, qkt_hbm, v_hbm, out_hbm,
          acc_sh, rI, cI, qaA, qaB, kbA, kbB, vbA, vbB, wbA, wbB,
          sgA, sgB, ssA, ssB, sidx):
        cid = lax.axis_index("c")
        sid = lax.axis_index("s")
        wid = cid * _NS + sid
        lane = lax.iota(jnp.int32, 16)
        rot8 = lane ^ 8
        hsel = lane >> 3
        zf = (lane * 0).astype(_F32)

        r0 = pl.multiple_of(wid * _NCH, 8)
        di1 = pltpu.async_copy(row_hbm.at[pl.ds(r0, _NCH)], rI, sidx)
        di2 = pltpu.async_copy(col_hbm.at[pl.ds(r0, _NCH)], cI, sidx)

        @pl.loop(0, _CH)
        def _(i):
            for k2 in range(5):
                wbA[i, pl.ds(16 * k2, 16)] = zf
        for m in range(5):
            pltpu.sync_copy(wbA, acc_sh.at[pl.ds(sid * 640 + m * _CH, _CH)])
        plsc.subcore_barrier()
        di1.wait()
        di2.wait()

        def issue(c, qa, kb, vb, sem):
            pltpu.async_copy(qkt_hbm.at[rI.at[c]], qa, sem)
            pltpu.async_copy(qkt_hbm.at[cI.at[c]], kb, sem)
            pltpu.async_copy(v_hbm.at[cI.at[c]], vb, sem)

        def drain(c, qa, kb, vb, sem):
            pltpu.make_async_copy(qkt_hbm.at[rI.at[c]], qa, sem).wait()
            pltpu.make_async_copy(qkt_hbm.at[cI.at[c]], kb, sem).wait()
            pltpu.make_async_copy(v_hbm.at[cI.at[c]], vb, sem).wait()

        def compute(qa, kb, vb, wb):
            @pl.loop(0, _CH)
            def _(i):
                a = qa[i, :]
                b = kb[i, :]
                e = jnp.exp(a * _dyn_gather(b, rot8))
                for k2 in range(4):
                    ev = _dyn_gather(e, hsel + 2 * k2)
                    wb[i, pl.ds(16 * k2, 16)] = vb[i, pl.ds(16 * k2, 16)] * ev
                wb[i, pl.ds(64, 16)] = jnp.where(lane < 8, e, 0.0)

        issue(0, qaA, kbA, vbA, sgA)
        issue(1, qaB, kbB, vbB, sgB)

        @pl.loop(0, _NCH // 2)
        def _(t):
            c0 = 2 * t
            c1 = 2 * t + 1
            drain(c0, qaA, kbA, vbA, sgA)
            compute(qaA, kbA, vbA, wbA)
            dsa = pltpu.async_copy(wbA, acc_sh.at[rI.at[c0]], ssA, add=True)

            @pl.when(t < _NCH // 2 - 1)
            def _():
                issue(c0 + 2, qaA, kbA, vbA, sgA)

            drain(c1, qaB, kbB, vbB, sgB)
            compute(qaB, kbB, vbB, wbB)
            dsb = pltpu.async_copy(wbB, acc_sh.at[rI.at[c1]], ssB, add=True)

            @pl.when(t < _NCH // 2 - 1)
            def _():
                issue(c1 + 2, qaB, kbB, vbB, sgB)

            dsa.wait()
            dsb.wait()

        plsc.subcore_barrier()
        pltpu.sync_copy(acc_sh.at[pl.ds(sid * 640, 640)],
                        out_hbm.at[pl.ds(cid * _NP + sid * 640, 640)])

    return k(rowp, colp, qkt, v0)


def _scl1(rowp, colp, q1t, k1t, v1):
    @functools.partial(
        pl.kernel,
        out_type=jax.ShapeDtypeStruct((2 * _NP, 64), _F32),
        mesh=_mesh(),
        compiler_params=_SC_PARAMS,
        scratch_types=[
            pltpu.VMEM_SHARED((_NP, 64), _F32),
            pltpu.VMEM((_NCH, _CH), jnp.int32),
            pltpu.VMEM((_NCH, _CH), jnp.int32),
            pltpu.VMEM((_NP,), _F32),
            pltpu.VMEM((_NP,), _F32),
            pltpu.VMEM((_CH, 48), _F32),
            pltpu.VMEM((_CH, 48), _F32),
            pltpu.VMEM((_CH, 64), _F32),
            pltpu.VMEM((_CH, 64), _F32),
            pltpu.SemaphoreType.DMA,
            pltpu.SemaphoreType.DMA,
            pltpu.SemaphoreType.DMA,
            pltpu.SemaphoreType.DMA,
            pltpu.SemaphoreType.DMA,
        ],
    )
    def k(row_hbm, col_hbm, q1_hbm, k1_hbm, v1_hbm, out_hbm,
          acc_sh, rI, cI, q1t_v, k1t_v, vbA, vbB, wbA, wbB,
          sgA, sgB, ssA, ssB, sidx):
        cid = lax.axis_index("c")
        sid = lax.axis_index("s")
        wid = cid * _NS + sid
        lane = lax.iota(jnp.int32, 16)
        zf = (lane * 0).astype(_F32)

        r0 = pl.multiple_of(wid * _NCH, 8)
        di1 = pltpu.async_copy(row_hbm.at[pl.ds(r0, _NCH)], rI, sidx)
        di2 = pltpu.async_copy(col_hbm.at[pl.ds(r0, _NCH)], cI, sidx)
        dq = pltpu.async_copy(q1_hbm, q1t_v, sidx)
        dk = pltpu.async_copy(k1_hbm, k1t_v, sidx)

        @pl.loop(0, _CH)
        def _(i):
            for k2 in range(4):
                wbA[i, pl.ds(16 * k2, 16)] = zf
        for m in range(5):
            pltpu.sync_copy(wbA, acc_sh.at[pl.ds(sid * 640 + m * _CH, _CH)])
        plsc.subcore_barrier()
        di1.wait()
        di2.wait()
        dq.wait()
        dk.wait()

        def issue(c, vb, sem):
            pltpu.async_copy(v1_hbm.at[cI.at[c]], vb, sem)

        def drain(c, vb, sem):
            pltpu.make_async_copy(v1_hbm.at[cI.at[c]], vb, sem).wait()

        def compute(c, vb, wb):
            @pl.loop(0, _CH // 16)
            def _(g):
                idxr = rI[c, pl.ds(g * 16, 16)]
                idxc = cI[c, pl.ds(g * 16, 16)]
                qg = plsc.load_gather(q1t_v, [idxr])
                kg = plsc.load_gather(k1t_v, [idxc])
                e1 = jnp.exp(qg * kg)
                for i in range(16):
                    sp = _dyn_gather(e1, lane * 0 + i)
                    ei = g * 16 + i
                    for k2 in range(3):
                        wb[ei, pl.ds(16 * k2, 16)] = (
                            vb[ei, pl.ds(16 * k2, 16)] * sp)
                    wb[ei, pl.ds(48, 16)] = jnp.where(lane < 1, sp, 0.0)

        issue(0, vbA, sgA)
        issue(1, vbB, sgB)

        @pl.loop(0, _NCH // 2)
        def _(t):
            c0 = 2 * t
            c1 = 2 * t + 1
            drain(c0, vbA, sgA)
            compute(c0, vbA, wbA)
            dsa = pltpu.async_copy(wbA, acc_sh.at[rI.at[c0]], ssA, add=True)

            @pl.when(t < _NCH // 2 - 1)
            def _():
                issue(c0 + 2, vbA, sgA)

            drain(c1, vbB, sgB)
            compute(c1, vbB, wbB)
            dsb = pltpu.async_copy(wbB, acc_sh.at[rI.at[c1]], ssB, add=True)

            @pl.when(t < _NCH // 2 - 1)
            def _():
                issue(c1 + 2, vbB, sgB)

            dsa.wait()
            dsb.wait()

        plsc.subcore_barrier()
        pltpu.sync_copy(acc_sh.at[pl.ds(sid * 640, 640)],
                        out_hbm.at[pl.ds(cid * _NP + sid * 640, 640)])

    return k(rowp, colp, q1t, k1t, v1)


def kernel(x, edge_index, edge_weight, qk0, qb0, kk0, kb0, k0, b0,
           qk1, qb1, kk1, kb1, k1, b1):
    del edge_weight
    xp = jnp.zeros((_NP, 128), _F32).at[:_N].set(x)
    row = edge_index[0].astype(jnp.int32)
    col = edge_index[1].astype(jnp.int32)
    pad = jnp.full((_EP - _E,), _N, jnp.int32)
    rowp = jnp.concatenate([row, pad]).reshape(_EP // _CH, _CH)
    colp = jnp.concatenate([col, pad]).reshape(_EP // _CH, _CH)
    p8 = jnp.kron(jnp.eye(8, dtype=_F32), jnp.ones((1, 8), _F32))

    qkt, v0, es0, outs0 = _tc1(xp, qk0, qb0.reshape(1, 8), kk0,
                               kb0.reshape(1, 8), k0, p8)
    accp = _scl0(rowp, colp, qkt, v0)
    k1p = jnp.zeros((64, 48), _F32).at[:, :40].set(k1)
    q1, k1c, v1, es1, outs1 = _tc2(accp, es0, outs0, p8, b0.reshape(1, 64),
                                   qk1, qb1.reshape(1, 1), kk1,
                                   kb1.reshape(1, 1), k1p)
    acc1p = _scl1(rowp, colp, q1.reshape(_NP), k1c.reshape(_NP), v1)
    b1p = jnp.zeros((1, 48), _F32).at[0, :40].set(b1)
    out = _tc3(acc1p, es1, outs1, b1p)
    return out[:_N, :40]


# speedup vs baseline: 84.2464x; 1.0635x over previous
import functools

import jax
import jax.numpy as jnp
from jax import lax
from jax.experimental import pallas as pl
from jax.experimental.pallas import tpu as pltpu
from jax.experimental.pallas import tpu_sc as plsc

_N = 10000
_NP = 10240
_E = 320000
_NC, _NS = 2, 16
_NW = _NC * _NS
_EPW = 10240
_EP = _NW * _EPW
_CH = 128
_NCH = _EPW // _CH
_F32 = jnp.float32


def _dyn_gather(v, idx):
    dnums = lax.GatherDimensionNumbers(
        offset_dims=(), collapsed_slice_dims=(0,), start_index_map=(0,))
    return lax.gather(v, idx[:, None], dnums, (1,),
                      mode=lax.GatherScatterMode.PROMISE_IN_BOUNDS)


def _tc1(xp, qk0, qb0, kk0, kb0, k0, p8):
    def body(x_ref, qk_ref, qb_ref, kk_ref, kb_ref, kv_ref, p_ref,
             qkt_ref, v_ref, es_ref, outs_ref):
        x = x_ref[...]
        q = jnp.maximum(jnp.dot(x, qk_ref[...], preferred_element_type=_F32)
                        + qb_ref[...], 0.0)
        k = jnp.maximum(jnp.dot(x, kk_ref[...], preferred_element_type=_F32)
                        + kb_ref[...], 0.0)
        v = jnp.dot(x, kv_ref[...], preferred_element_type=_F32)
        es = jnp.exp(q * k)
        es64 = jnp.dot(es, p_ref[...], preferred_element_type=_F32)
        qkt_ref[...] = jnp.concatenate([q, k], axis=1)
        v_ref[...] = v
        es_ref[...] = es
        outs_ref[...] = es64 * v
    return pl.pallas_call(
        body,
        out_shape=(
            jax.ShapeDtypeStruct((_NP, 16), _F32),
            jax.ShapeDtypeStruct((_NP, 64), _F32),
            jax.ShapeDtypeStruct((_NP, 8), _F32),
            jax.ShapeDtypeStruct((_NP, 64), _F32),
        ),
    )(xp, qk0, qb0, kk0, kb0, k0, p8)


def _tc2(accp, es0, outs0, p8, b0, qk1, qb1, kk1, kb1, k1p):
    def body(a_ref, es_ref, os_ref, p_ref, b0_ref,
             qk_ref, qb_ref, kk_ref, kb_ref, kv_ref,
             q1_ref, k1_ref, v1_ref, es1_ref, outs1_ref):
        acc = a_ref[0:_NP, :] + a_ref[_NP:2 * _NP, :]
        s = acc[:, 64:72] + es_ref[...]
        s64 = jnp.dot(s, p_ref[...], preferred_element_type=_F32)
        h = jnp.maximum((acc[:, 0:64] + os_ref[...]) / s64 + b0_ref[...], 0.0)
        q1 = jnp.maximum(jnp.dot(h, qk_ref[...], preferred_element_type=_F32)
                         + qb_ref[...], 0.0)
        k1 = jnp.maximum(jnp.dot(h, kk_ref[...], preferred_element_type=_F32)
                         + kb_ref[...], 0.0)
        v1 = jnp.dot(h, kv_ref[...], preferred_element_type=_F32)
        es1 = jnp.exp(q1 * k1)
        q1_ref[...] = q1
        k1_ref[...] = k1
        v1_ref[...] = v1
        es1_ref[...] = es1
        outs1_ref[...] = es1 * v1
    return pl.pallas_call(
        body,
        out_shape=(
            jax.ShapeDtypeStruct((_NP, 1), _F32),
            jax.ShapeDtypeStruct((_NP, 1), _F32),
            jax.ShapeDtypeStruct((_NP, 48), _F32),
            jax.ShapeDtypeStruct((_NP, 1), _F32),
            jax.ShapeDtypeStruct((_NP, 48), _F32),
        ),
    )(accp, es0, outs0, p8, b0, qk1, qb1, kk1, kb1, k1p)


def _tc3(acc1p, es1, outs1, b1p):
    def body(a_ref, es_ref, os_ref, b_ref, o_ref):
        acc = a_ref[0:_NP, :] + a_ref[_NP:2 * _NP, :]
        s1 = acc[:, 48:49] + es_ref[...]
        o_ref[...] = (acc[:, 0:48] + os_ref[...]) / s1 + b_ref[...]
    return pl.pallas_call(
        body,
        out_shape=jax.ShapeDtypeStruct((_NP, 48), _F32),
    )(acc1p, es1, outs1, b1p)


def _mesh():
    return plsc.VectorSubcoreMesh(core_axis_name="c", subcore_axis_name="s",
                                  num_cores=_NC, num_subcores=_NS)


_SC_PARAMS = pltpu.CompilerParams(use_tc_tiling_on_sc=False,
                                  needs_layout_passes=False)


def _scl0(rowp, colp, qkt, v0):
    @functools.partial(
        pl.kernel,
        out_type=jax.ShapeDtypeStruct((2 * _NP, 80), _F32),
        mesh=_mesh(),
        compiler_params=_SC_PARAMS,
        scratch_types=[
            pltpu.VMEM_SHARED((_NP, 80), _F32),
            pltpu.VMEM((_NCH, _CH), jnp.int32),
            pltpu.VMEM((_NCH, _CH), jnp.int32),
            pltpu.VMEM((_CH, 16), _F32),
            pltpu.VMEM((_CH, 16), _F32),
            pltpu.VMEM((_CH, 16), _F32),
            pltpu.VMEM((_CH, 16), _F32),
            pltpu.VMEM((_CH, 64), _F32),
            pltpu.VMEM((_CH, 64), _F32),
            pltpu.VMEM((_CH, 80), _F32),
            pltpu.VMEM((_CH, 80), _F32),
            pltpu.SemaphoreType.DMA,
            pltpu.SemaphoreType.DMA,
            pltpu.SemaphoreType.DMA,
            pltpu.SemaphoreType.DMA,
            pltpu.SemaphoreType.DMA,
        ],
    )
    def k(row_hbm, col_hbm, qkt_hbm, v_hbm, out_hbm,
          acc_sh, rI, cI, qaA, qaB, kbA, kbB, vbA, vbB, wbA, wbB,
          sgA, sgB, ssA, ssB, sidx):
        cid = lax.axis_index("c")
        sid = lax.axis_index("s")
        wid = cid * _NS + sid
        lane = lax.iota(jnp.int32, 16)
        rot8 = lane ^ 8
        hsel = lane >> 3
        zf = (lane * 0).astype(_F32)

        r0 = pl.multiple_of(wid * _NCH, 8)
        di1 = pltpu.async_copy(row_hbm.at[pl.ds(r0, _NCH)], rI, sidx)
        di2 = pltpu.async_copy(col_hbm.at[pl.ds(r0, _NCH)], cI, sidx)

        @pl.loop(0, _CH)
        def _(i):
            for k2 in range(5):
                wbA[i, pl.ds(16 * k2, 16)] = zf
        for m in range(5):
            pltpu.sync_copy(wbA, acc_sh.at[pl.ds(sid * 640 + m * _CH, _CH)])
        plsc.subcore_barrier()
        di1.wait()
        di2.wait()

        def issue(c, qa, kb, vb, sem):
            pltpu.async_copy(qkt_hbm.at[rI.at[c]], qa, sem)
            pltpu.async_copy(qkt_hbm.at[cI.at[c]], kb, sem)
            pltpu.async_copy(v_hbm.at[cI.at[c]], vb, sem)

        def drain(c, qa, kb, vb, sem):
            pltpu.make_async_copy(qkt_hbm.at[rI.at[c]], qa, sem).wait()
            pltpu.make_async_copy(qkt_hbm.at[cI.at[c]], kb, sem).wait()
            pltpu.make_async_copy(v_hbm.at[cI.at[c]], vb, sem).wait()

        def compute(qa, kb, vb, wb):
            @pl.loop(0, _CH)
            def _(i):
                a = qa[i, :]
                b = kb[i, :]
                e = jnp.exp(a * _dyn_gather(b, rot8))
                for k2 in range(4):
                    ev = _dyn_gather(e, hsel + 2 * k2)
                    wb[i, pl.ds(16 * k2, 16)] = vb[i, pl.ds(16 * k2, 16)] * ev
                wb[i, pl.ds(64, 16)] = jnp.where(lane < 8, e, 0.0)

        issue(0, qaA, kbA, vbA, sgA)
        issue(1, qaB, kbB, vbB, sgB)

        @pl.loop(0, _NCH // 2)
        def _(t):
            c0 = 2 * t
            c1 = 2 * t + 1
            drain(c0, qaA, kbA, vbA, sgA)
            compute(qaA, kbA, vbA, wbA)
            dsa = pltpu.async_copy(wbA, acc_sh.at[rI.at[c0]], ssA, add=True)

            @pl.when(t < _NCH // 2 - 1)
            def _():
                issue(c0 + 2, qaA, kbA, vbA, sgA)

            drain(c1, qaB, kbB, vbB, sgB)
            compute(qaB, kbB, vbB, wbB)
            dsb = pltpu.async_copy(wbB, acc_sh.at[rI.at[c1]], ssB, add=True)

            @pl.when(t < _NCH // 2 - 1)
            def _():
                issue(c1 + 2, qaB, kbB, vbB, sgB)

            dsa.wait()
            dsb.wait()

        plsc.subcore_barrier()
        pltpu.sync_copy(acc_sh.at[pl.ds(sid * 640, 640)],
                        out_hbm.at[pl.ds(cid * _NP + sid * 640, 640)])

    return k(rowp, colp, qkt, v0)


def _scl1(rowp, colp, q1t, k1t, v1):
    @functools.partial(
        pl.kernel,
        out_type=jax.ShapeDtypeStruct((2 * _NP, 64), _F32),
        mesh=_mesh(),
        compiler_params=_SC_PARAMS,
        scratch_types=[
            pltpu.VMEM_SHARED((_NP, 64), _F32),
            pltpu.VMEM((_NCH, _CH), jnp.int32),
            pltpu.VMEM((_NCH, _CH), jnp.int32),
            pltpu.VMEM((_NP,), _F32),
            pltpu.VMEM((_NP,), _F32),
            pltpu.VMEM((_CH, 48), _F32),
            pltpu.VMEM((_CH, 48), _F32),
            pltpu.VMEM((_CH, 64), _F32),
            pltpu.VMEM((_CH, 64), _F32),
            pltpu.SemaphoreType.DMA,
            pltpu.SemaphoreType.DMA,
            pltpu.SemaphoreType.DMA,
            pltpu.SemaphoreType.DMA,
            pltpu.SemaphoreType.DMA,
        ],
    )
    def k(row_hbm, col_hbm, q1_hbm, k1_hbm, v1_hbm, out_hbm,
          acc_sh, rI, cI, q1t_v, k1t_v, vbA, vbB, wbA, wbB,
          sgA, sgB, ssA, ssB, sidx):
        cid = lax.axis_index("c")
        sid = lax.axis_index("s")
        wid = cid * _NS + sid
        lane = lax.iota(jnp.int32, 16)
        zf = (lane * 0).astype(_F32)

        r0 = pl.multiple_of(wid * _NCH, 8)
        di1 = pltpu.async_copy(row_hbm.at[pl.ds(r0, _NCH)], rI, sidx)
        di2 = pltpu.async_copy(col_hbm.at[pl.ds(r0, _NCH)], cI, sidx)
        dq = pltpu.async_copy(q1_hbm, q1t_v, sidx)
        dk = pltpu.async_copy(k1_hbm, k1t_v, sidx)

        @pl.loop(0, _CH)
        def _(i):
            for k2 in range(4):
                wbA[i, pl.ds(16 * k2, 16)] = zf
        for m in range(5):
            pltpu.sync_copy(wbA, acc_sh.at[pl.ds(sid * 640 + m * _CH, _CH)])
        plsc.subcore_barrier()
        di1.wait()
        di2.wait()
        dq.wait()
        dk.wait()

        def issue(c, vb, sem):
            pltpu.async_copy(v1_hbm.at[cI.at[c]], vb, sem)

        def drain(c, vb, sem):
            pltpu.make_async_copy(v1_hbm.at[cI.at[c]], vb, sem).wait()

        def compute(c, vb, wb):
            @pl.loop(0, _CH // 16)
            def _(g):
                idxr = rI[c, pl.ds(g * 16, 16)]
                idxc = cI[c, pl.ds(g * 16, 16)]
                qg = plsc.load_gather(q1t_v, [idxr])
                kg = plsc.load_gather(k1t_v, [idxc])
                e1 = jnp.exp(qg * kg)
                for i in range(16):
                    sp = _dyn_gather(e1, lane * 0 + i)
                    ei = g * 16 + i
                    for k2 in range(3):
                        wb[ei, pl.ds(16 * k2, 16)] = (
                            vb[ei, pl.ds(16 * k2, 16)] * sp)
                    wb[ei, pl.ds(48, 16)] = jnp.where(lane < 1, sp, 0.0)

        issue(0, vbA, sgA)
        issue(1, vbB, sgB)

        @pl.loop(0, _NCH // 2)
        def _(t):
            c0 = 2 * t
            c1 = 2 * t + 1
            drain(c0, vbA, sgA)
            compute(c0, vbA, wbA)
            dsa = pltpu.async_copy(wbA, acc_sh.at[rI.at[c0]], ssA, add=True)

            @pl.when(t < _NCH // 2 - 1)
            def _():
                issue(c0 + 2, vbA, sgA)

            drain(c1, vbB, sgB)
            compute(c1, vbB, wbB)
            dsb = pltpu.async_copy(wbB, acc_sh.at[rI.at[c1]], ssB, add=True)

            @pl.when(t < _NCH // 2 - 1)
            def _():
                issue(c1 + 2, vbB, sgB)

            dsa.wait()
            dsb.wait()

        plsc.subcore_barrier()
        pltpu.sync_copy(acc_sh.at[pl.ds(sid * 640, 640)],
                        out_hbm.at[pl.ds(cid * _NP + sid * 640, 640)])

    return k(rowp, colp, q1t, k1t, v1)


def kernel(x, edge_index, edge_weight, qk0, qb0, kk0, kb0, k0, b0,
           qk1, qb1, kk1, kb1, k1, b1):
    del edge_weight
    xp = jnp.zeros((_NP, 128), _F32).at[:_N].set(x)
    row = edge_index[0].astype(jnp.int32)
    col = edge_index[1].astype(jnp.int32)
    pad = _N + (jnp.arange(_EP - _E, dtype=jnp.int32) % (_NP - _N))
    rowp = jnp.concatenate([row, pad]).reshape(_EP // _CH, _CH)
    colp = jnp.concatenate([col, pad]).reshape(_EP // _CH, _CH)
    p8 = jnp.kron(jnp.eye(8, dtype=_F32), jnp.ones((1, 8), _F32))

    qkt, v0, es0, outs0 = _tc1(xp, qk0, qb0.reshape(1, 8), kk0,
                               kb0.reshape(1, 8), k0, p8)
    accp = _scl0(rowp, colp, qkt, v0)
    k1p = jnp.zeros((64, 48), _F32).at[:, :40].set(k1)
    q1, k1c, v1, es1, outs1 = _tc2(accp, es0, outs0, p8, b0.reshape(1, 64),
                                   qk1, qb1.reshape(1, 1), kk1,
                                   kb1.reshape(1, 1), k1p)
    acc1p = _scl1(rowp, colp, q1.reshape(_NP), k1c.reshape(_NP), v1)
    b1p = jnp.zeros((1, 48), _F32).at[0, :40].set(b1)
    out = _tc3(acc1p, es1, outs1, b1p)
    return out[:_N, :40]


# speedup vs baseline: 154.1063x; 1.8292x over previous
import functools

import jax
import jax.numpy as jnp
from jax import lax
from jax.experimental import pallas as pl
from jax.experimental.pallas import tpu as pltpu
from jax.experimental.pallas import tpu_sc as plsc

_N = 10000
_NP = 10240
_E = 320000
_NC, _NS = 2, 16
_NW = _NC * _NS
_EPW = 10240
_EP = _NW * _EPW
_CH = 128
_NCH = _EPW // _CH
_F32 = jnp.float32


def _dyn_gather(v, idx):
    dnums = lax.GatherDimensionNumbers(
        offset_dims=(), collapsed_slice_dims=(0,), start_index_map=(0,))
    return lax.gather(v, idx[:, None], dnums, (1,),
                      mode=lax.GatherScatterMode.PROMISE_IN_BOUNDS)


def _tc1(xp, qk0, qb0, kk0, kb0, k0, p8):
    def body(x_ref, qk_ref, qb_ref, kk_ref, kb_ref, kv_ref, p_ref,
             qkt_ref, v_ref, es_ref, outs_ref):
        x = x_ref[...]
        q = jnp.maximum(jnp.dot(x, qk_ref[...], preferred_element_type=_F32)
                        + qb_ref[...], 0.0)
        k = jnp.maximum(jnp.dot(x, kk_ref[...], preferred_element_type=_F32)
                        + kb_ref[...], 0.0)
        v = jnp.dot(x, kv_ref[...], preferred_element_type=_F32)
        es = jnp.exp(q * k)
        es64 = jnp.dot(es, p_ref[...], preferred_element_type=_F32)
        qkt_ref[...] = jnp.concatenate([q, k], axis=1)
        v_ref[...] = v
        es_ref[...] = es
        outs_ref[...] = es64 * v
    return pl.pallas_call(
        body,
        out_shape=(
            jax.ShapeDtypeStruct((_NP, 16), _F32),
            jax.ShapeDtypeStruct((_NP, 64), _F32),
            jax.ShapeDtypeStruct((_NP, 8), _F32),
            jax.ShapeDtypeStruct((_NP, 64), _F32),
        ),
    )(xp, qk0, qb0, kk0, kb0, k0, p8)


def _tc2(accp, es0, outs0, p8, b0, qk1, qb1, kk1, kb1, k1p):
    def body(a_ref, es_ref, os_ref, p_ref, b0_ref,
             qk_ref, qb_ref, kk_ref, kb_ref, kv_ref,
             q1_ref, k1_ref, v1_ref, es1_ref, outs1_ref):
        acc = a_ref[0:_NP, :] + a_ref[_NP:2 * _NP, :]
        s = acc[:, 64:72] + es_ref[...]
        s64 = jnp.dot(s, p_ref[...], preferred_element_type=_F32)
        h = jnp.maximum((acc[:, 0:64] + os_ref[...]) / s64 + b0_ref[...], 0.0)
        q1 = jnp.maximum(jnp.dot(h, qk_ref[...], preferred_element_type=_F32)
                         + qb_ref[...], 0.0)
        k1 = jnp.maximum(jnp.dot(h, kk_ref[...], preferred_element_type=_F32)
                         + kb_ref[...], 0.0)
        v1 = jnp.dot(h, kv_ref[...], preferred_element_type=_F32)
        es1 = jnp.exp(q1 * k1)
        q1_ref[...] = q1
        k1_ref[...] = k1
        v1_ref[...] = v1
        es1_ref[...] = es1
        outs1_ref[...] = es1 * v1
    return pl.pallas_call(
        body,
        out_shape=(
            jax.ShapeDtypeStruct((_NP, 1), _F32),
            jax.ShapeDtypeStruct((_NP, 1), _F32),
            jax.ShapeDtypeStruct((_NP, 48), _F32),
            jax.ShapeDtypeStruct((_NP, 1), _F32),
            jax.ShapeDtypeStruct((_NP, 48), _F32),
        ),
    )(accp, es0, outs0, p8, b0, qk1, qb1, kk1, kb1, k1p)


def _tc3(acc1p, es1, outs1, b1p):
    def body(a_ref, es_ref, os_ref, b_ref, o_ref):
        acc = a_ref[0:_NP, :] + a_ref[_NP:2 * _NP, :]
        s1 = acc[:, 48:49] + es_ref[...]
        o_ref[...] = (acc[:, 0:48] + os_ref[...]) / s1 + b_ref[...]
    return pl.pallas_call(
        body,
        out_shape=jax.ShapeDtypeStruct((_NP, 48), _F32),
    )(acc1p, es1, outs1, b1p)


def _mesh():
    return plsc.VectorSubcoreMesh(core_axis_name="c", subcore_axis_name="s",
                                  num_cores=_NC, num_subcores=_NS)


_SC_PARAMS = pltpu.CompilerParams(use_tc_tiling_on_sc=False,
                                  needs_layout_passes=False)


def _scl0(rowp, colp, qkt, v0):
    @functools.partial(
        pl.kernel,
        out_type=jax.ShapeDtypeStruct((2 * _NP, 80), _F32),
        mesh=_mesh(),
        compiler_params=_SC_PARAMS,
        scratch_types=[
            pltpu.VMEM_SHARED((_NP, 80), _F32),
            pltpu.VMEM((_NCH, _CH), jnp.int32),
            pltpu.VMEM((_NCH, _CH), jnp.int32),
            pltpu.VMEM((_CH, 16), _F32),
            pltpu.VMEM((_CH, 16), _F32),
            pltpu.VMEM((_CH, 16), _F32),
            pltpu.VMEM((_CH, 16), _F32),
            pltpu.VMEM((_CH, 64), _F32),
            pltpu.VMEM((_CH, 64), _F32),
            pltpu.VMEM((_CH, 80), _F32),
            pltpu.VMEM((_CH, 80), _F32),
            pltpu.SemaphoreType.DMA,
            pltpu.SemaphoreType.DMA,
            pltpu.SemaphoreType.DMA,
            pltpu.SemaphoreType.DMA,
            pltpu.SemaphoreType.DMA,
        ],
    )
    def k(row_hbm, col_hbm, qkt_hbm, v_hbm, out_hbm,
          acc_sh, rI, cI, qaA, qaB, kbA, kbB, vbA, vbB, wbA, wbB,
          sgA, sgB, ssA, ssB, sidx):
        cid = lax.axis_index("c")
        sid = lax.axis_index("s")
        wid = cid * _NS + sid
        lane = lax.iota(jnp.int32, 16)
        rot8 = lane ^ 8
        hsel = lane >> 3
        zf = (lane * 0).astype(_F32)

        r0 = pl.multiple_of(wid * _NCH, 8)
        di1 = pltpu.async_copy(row_hbm.at[pl.ds(r0, _NCH)], rI, sidx)
        di2 = pltpu.async_copy(col_hbm.at[pl.ds(r0, _NCH)], cI, sidx)

        @pl.loop(0, _CH)
        def _(i):
            for k2 in range(5):
                wbA[i, pl.ds(16 * k2, 16)] = zf
        for m in range(5):
            pltpu.sync_copy(wbA, acc_sh.at[pl.ds(sid * 640 + m * _CH, _CH)])
        plsc.subcore_barrier()
        di1.wait()
        di2.wait()

        def issue(c, qa, kb, vb, sem):
            pltpu.async_copy(qkt_hbm.at[rI.at[c]], qa, sem)
            pltpu.async_copy(qkt_hbm.at[cI.at[c]], kb, sem)
            pltpu.async_copy(v_hbm.at[cI.at[c]], vb, sem)

        def drain(c, qa, kb, vb, sem):
            pltpu.make_async_copy(qkt_hbm.at[rI.at[c]], qa, sem).wait()
            pltpu.make_async_copy(qkt_hbm.at[cI.at[c]], kb, sem).wait()
            pltpu.make_async_copy(v_hbm.at[cI.at[c]], vb, sem).wait()

        def compute(qa, kb, vb, wb):
            @plsc.parallel_loop(0, _CH, unroll=4)
            def _(i):
                a = qa[i, :]
                b = kb[i, :]
                e = jnp.exp(a * _dyn_gather(b, rot8))
                for k2 in range(4):
                    ev = _dyn_gather(e, hsel + 2 * k2)
                    wb[i, pl.ds(16 * k2, 16)] = vb[i, pl.ds(16 * k2, 16)] * ev
                wb[i, pl.ds(64, 16)] = jnp.where(lane < 8, e, 0.0)

        issue(0, qaA, kbA, vbA, sgA)
        issue(1, qaB, kbB, vbB, sgB)

        @pl.loop(0, _NCH // 2)
        def _(t):
            c0 = 2 * t
            c1 = 2 * t + 1
            drain(c0, qaA, kbA, vbA, sgA)
            compute(qaA, kbA, vbA, wbA)
            dsa = pltpu.async_copy(wbA, acc_sh.at[rI.at[c0]], ssA, add=True)

            @pl.when(t < _NCH // 2 - 1)
            def _():
                issue(c0 + 2, qaA, kbA, vbA, sgA)

            drain(c1, qaB, kbB, vbB, sgB)
            compute(qaB, kbB, vbB, wbB)
            dsb = pltpu.async_copy(wbB, acc_sh.at[rI.at[c1]], ssB, add=True)

            @pl.when(t < _NCH // 2 - 1)
            def _():
                issue(c1 + 2, qaB, kbB, vbB, sgB)

            dsa.wait()
            dsb.wait()

        plsc.subcore_barrier()
        pltpu.sync_copy(acc_sh.at[pl.ds(sid * 640, 640)],
                        out_hbm.at[pl.ds(cid * _NP + sid * 640, 640)])

    return k(rowp, colp, qkt, v0)


def _scl1(rowp, colp, q1t, k1t, v1):
    @functools.partial(
        pl.kernel,
        out_type=jax.ShapeDtypeStruct((2 * _NP, 64), _F32),
        mesh=_mesh(),
        compiler_params=_SC_PARAMS,
        scratch_types=[
            pltpu.VMEM_SHARED((_NP, 64), _F32),
            pltpu.VMEM((_NCH, _CH), jnp.int32),
            pltpu.VMEM((_NCH, _CH), jnp.int32),
            pltpu.VMEM((_NP,), _F32),
            pltpu.VMEM((_NP,), _F32),
            pltpu.VMEM((_CH, 48), _F32),
            pltpu.VMEM((_CH, 48), _F32),
            pltpu.VMEM((_CH, 64), _F32),
            pltpu.VMEM((_CH, 64), _F32),
            pltpu.SemaphoreType.DMA,
            pltpu.SemaphoreType.DMA,
            pltpu.SemaphoreType.DMA,
            pltpu.SemaphoreType.DMA,
            pltpu.SemaphoreType.DMA,
        ],
    )
    def k(row_hbm, col_hbm, q1_hbm, k1_hbm, v1_hbm, out_hbm,
          acc_sh, rI, cI, q1t_v, k1t_v, vbA, vbB, wbA, wbB,
          sgA, sgB, ssA, ssB, sidx):
        cid = lax.axis_index("c")
        sid = lax.axis_index("s")
        wid = cid * _NS + sid
        lane = lax.iota(jnp.int32, 16)
        zf = (lane * 0).astype(_F32)

        r0 = pl.multiple_of(wid * _NCH, 8)
        di1 = pltpu.async_copy(row_hbm.at[pl.ds(r0, _NCH)], rI, sidx)
        di2 = pltpu.async_copy(col_hbm.at[pl.ds(r0, _NCH)], cI, sidx)
        dq = pltpu.async_copy(q1_hbm, q1t_v, sidx)
        dk = pltpu.async_copy(k1_hbm, k1t_v, sidx)

        @pl.loop(0, _CH)
        def _(i):
            for k2 in range(4):
                wbA[i, pl.ds(16 * k2, 16)] = zf
        for m in range(5):
            pltpu.sync_copy(wbA, acc_sh.at[pl.ds(sid * 640 + m * _CH, _CH)])
        plsc.subcore_barrier()
        di1.wait()
        di2.wait()
        dq.wait()
        dk.wait()

        def issue(c, vb, sem):
            pltpu.async_copy(v1_hbm.at[cI.at[c]], vb, sem)

        def drain(c, vb, sem):
            pltpu.make_async_copy(v1_hbm.at[cI.at[c]], vb, sem).wait()

        def compute(c, vb, wb):
            @plsc.parallel_loop(0, _CH // 16, unroll=2)
            def _(g):
                idxr = rI[c, pl.ds(g * 16, 16)]
                idxc = cI[c, pl.ds(g * 16, 16)]
                qg = plsc.load_gather(q1t_v, [idxr])
                kg = plsc.load_gather(k1t_v, [idxc])
                e1 = jnp.exp(qg * kg)
                for i in range(16):
                    sp = _dyn_gather(e1, lane * 0 + i)
                    ei = g * 16 + i
                    for k2 in range(3):
                        wb[ei, pl.ds(16 * k2, 16)] = (
                            vb[ei, pl.ds(16 * k2, 16)] * sp)
                    wb[ei, pl.ds(48, 16)] = jnp.where(lane < 1, sp, 0.0)

        issue(0, vbA, sgA)
        issue(1, vbB, sgB)

        @pl.loop(0, _NCH // 2)
        def _(t):
            c0 = 2 * t
            c1 = 2 * t + 1
            drain(c0, vbA, sgA)
            compute(c0, vbA, wbA)
            dsa = pltpu.async_copy(wbA, acc_sh.at[rI.at[c0]], ssA, add=True)

            @pl.when(t < _NCH // 2 - 1)
            def _():
                issue(c0 + 2, vbA, sgA)

            drain(c1, vbB, sgB)
            compute(c1, vbB, wbB)
            dsb = pltpu.async_copy(wbB, acc_sh.at[rI.at[c1]], ssB, add=True)

            @pl.when(t < _NCH // 2 - 1)
            def _():
                issue(c1 + 2, vbB, sgB)

            dsa.wait()
            dsb.wait()

        plsc.subcore_barrier()
        pltpu.sync_copy(acc_sh.at[pl.ds(sid * 640, 640)],
                        out_hbm.at[pl.ds(cid * _NP + sid * 640, 640)])

    return k(rowp, colp, q1t, k1t, v1)


def kernel(x, edge_index, edge_weight, qk0, qb0, kk0, kb0, k0, b0,
           qk1, qb1, kk1, kb1, k1, b1):
    del edge_weight
    xp = jnp.zeros((_NP, 128), _F32).at[:_N].set(x)
    row = edge_index[0].astype(jnp.int32)
    col = edge_index[1].astype(jnp.int32)
    pad = _N + (jnp.arange(_EP - _E, dtype=jnp.int32) % (_NP - _N))
    rowp = jnp.concatenate([row, pad]).reshape(_EP // _CH, _CH)
    colp = jnp.concatenate([col, pad]).reshape(_EP // _CH, _CH)
    p8 = jnp.kron(jnp.eye(8, dtype=_F32), jnp.ones((1, 8), _F32))

    qkt, v0, es0, outs0 = _tc1(xp, qk0, qb0.reshape(1, 8), kk0,
                               kb0.reshape(1, 8), k0, p8)
    accp = _scl0(rowp, colp, qkt, v0)
    k1p = jnp.zeros((64, 48), _F32).at[:, :40].set(k1)
    q1, k1c, v1, es1, outs1 = _tc2(accp, es0, outs0, p8, b0.reshape(1, 64),
                                   qk1, qb1.reshape(1, 1), kk1,
                                   kb1.reshape(1, 1), k1p)
    acc1p = _scl1(rowp, colp, q1.reshape(_NP), k1c.reshape(_NP), v1)
    b1p = jnp.zeros((1, 48), _F32).at[0, :40].set(b1)
    out = _tc3(acc1p, es1, outs1, b1p)
    return out[:_N, :40]


# speedup vs baseline: 185.0529x; 1.2008x over previous
import functools

import jax
import jax.numpy as jnp
from jax import lax
from jax.experimental import pallas as pl
from jax.experimental.pallas import tpu as pltpu
from jax.experimental.pallas import tpu_sc as plsc

_N = 10000
_NP = 10240
_E = 320000
_NC, _NS = 2, 16
_NW = _NC * _NS
_EPW = 10240
_EP = _NW * _EPW
_CH = 128
_NCH = _EPW // _CH
_F32 = jnp.float32


def _dyn_gather(v, idx):
    dnums = lax.GatherDimensionNumbers(
        offset_dims=(), collapsed_slice_dims=(0,), start_index_map=(0,))
    return lax.gather(v, idx[:, None], dnums, (1,),
                      mode=lax.GatherScatterMode.PROMISE_IN_BOUNDS)


def _tc1(xp, qk0, qb0, kk0, kb0, k0, p8):
    def body(x_ref, qk_ref, qb_ref, kk_ref, kb_ref, kv_ref, p_ref,
             qkt_ref, v_ref, es_ref, outs_ref):
        x = x_ref[...]
        q = jnp.maximum(jnp.dot(x, qk_ref[...], preferred_element_type=_F32)
                        + qb_ref[...], 0.0)
        k = jnp.maximum(jnp.dot(x, kk_ref[...], preferred_element_type=_F32)
                        + kb_ref[...], 0.0)
        v = jnp.dot(x, kv_ref[...], preferred_element_type=_F32)
        es = jnp.exp(q * k)
        es64 = jnp.dot(es, p_ref[...], preferred_element_type=_F32)
        qkt_ref[...] = jnp.concatenate([q, k], axis=1)
        v_ref[...] = v
        es_ref[...] = es
        outs_ref[...] = es64 * v
    return pl.pallas_call(
        body,
        out_shape=(
            jax.ShapeDtypeStruct((_NP, 16), _F32),
            jax.ShapeDtypeStruct((_NP, 64), _F32),
            jax.ShapeDtypeStruct((_NP, 8), _F32),
            jax.ShapeDtypeStruct((_NP, 64), _F32),
        ),
    )(xp, qk0, qb0, kk0, kb0, k0, p8)


def _tc2(accp, es0, outs0, p8, b0, qk1, qb1, kk1, kb1, k1p):
    def body(a_ref, es_ref, os_ref, p_ref, b0_ref,
             qk_ref, qb_ref, kk_ref, kb_ref, kv_ref,
             q1_ref, k1_ref, v1_ref, es1_ref, outs1_ref):
        acc = a_ref[0:_NP, :] + a_ref[_NP:2 * _NP, :]
        s = acc[:, 64:72] + es_ref[...]
        s64 = jnp.dot(s, p_ref[...], preferred_element_type=_F32)
        h = jnp.maximum((acc[:, 0:64] + os_ref[...]) / s64 + b0_ref[...], 0.0)
        q1 = jnp.maximum(jnp.dot(h, qk_ref[...], preferred_element_type=_F32)
                         + qb_ref[...], 0.0)
        k1 = jnp.maximum(jnp.dot(h, kk_ref[...], preferred_element_type=_F32)
                         + kb_ref[...], 0.0)
        v1 = jnp.dot(h, kv_ref[...], preferred_element_type=_F32)
        es1 = jnp.exp(q1 * k1)
        q1_ref[...] = q1
        k1_ref[...] = k1
        v1_ref[...] = v1
        es1_ref[...] = es1
        outs1_ref[...] = es1 * v1
    return pl.pallas_call(
        body,
        out_shape=(
            jax.ShapeDtypeStruct((_NP, 1), _F32),
            jax.ShapeDtypeStruct((_NP, 1), _F32),
            jax.ShapeDtypeStruct((_NP, 48), _F32),
            jax.ShapeDtypeStruct((_NP, 1), _F32),
            jax.ShapeDtypeStruct((_NP, 48), _F32),
        ),
    )(accp, es0, outs0, p8, b0, qk1, qb1, kk1, kb1, k1p)


def _tc3(acc1p, es1, outs1, b1p):
    def body(a_ref, es_ref, os_ref, b_ref, o_ref):
        acc = a_ref[0:_NP, :] + a_ref[_NP:2 * _NP, :]
        s1 = acc[:, 48:49] + es_ref[...]
        o_ref[...] = (acc[:, 0:48] + os_ref[...]) / s1 + b_ref[...]
    return pl.pallas_call(
        body,
        out_shape=jax.ShapeDtypeStruct((_NP, 48), _F32),
    )(acc1p, es1, outs1, b1p)


def _mesh():
    return plsc.VectorSubcoreMesh(core_axis_name="c", subcore_axis_name="s",
                                  num_cores=_NC, num_subcores=_NS)


_SC_PARAMS = pltpu.CompilerParams(use_tc_tiling_on_sc=False,
                                  needs_layout_passes=False)


def _scl0(rowp, colp, qkt, v0):
    @functools.partial(
        pl.kernel,
        out_type=jax.ShapeDtypeStruct((2 * _NP, 80), _F32),
        mesh=_mesh(),
        compiler_params=_SC_PARAMS,
        scratch_types=[
            pltpu.VMEM_SHARED((_NP, 80), _F32),
            pltpu.VMEM((_NCH, _CH), jnp.int32),
            pltpu.VMEM((_NCH, _CH), jnp.int32),
            pltpu.VMEM((_CH, 16), _F32),
            pltpu.VMEM((_CH, 16), _F32),
            pltpu.VMEM((_CH, 16), _F32),
            pltpu.VMEM((_CH, 16), _F32),
            pltpu.VMEM((_CH, 64), _F32),
            pltpu.VMEM((_CH, 64), _F32),
            pltpu.VMEM((_CH, 80), _F32),
            pltpu.VMEM((_CH, 80), _F32),
            pltpu.SemaphoreType.DMA,
            pltpu.SemaphoreType.DMA,
            pltpu.SemaphoreType.DMA,
            pltpu.SemaphoreType.DMA,
            pltpu.SemaphoreType.DMA,
        ],
    )
    def k(row_hbm, col_hbm, qkt_hbm, v_hbm, out_hbm,
          acc_sh, rI, cI, qaA, qaB, kbA, kbB, vbA, vbB, wbA, wbB,
          sgA, sgB, ssA, ssB, sidx):
        cid = lax.axis_index("c")
        sid = lax.axis_index("s")
        wid = cid * _NS + sid
        lane = lax.iota(jnp.int32, 16)
        rot8 = lane ^ 8
        hsel = lane >> 3
        zf = (lane * 0).astype(_F32)

        r0 = pl.multiple_of(wid * _NCH, 8)
        di1 = pltpu.async_copy(row_hbm.at[pl.ds(r0, _NCH)], rI, sidx)
        di2 = pltpu.async_copy(col_hbm.at[pl.ds(r0, _NCH)], cI, sidx)

        @pl.loop(0, _CH)
        def _(i):
            for k2 in range(5):
                wbA[i, pl.ds(16 * k2, 16)] = zf
        for m in range(5):
            pltpu.sync_copy(wbA, acc_sh.at[pl.ds(sid * 640 + m * _CH, _CH)])
        plsc.subcore_barrier()
        di1.wait()
        di2.wait()

        def issue(c, qa, kb, vb, sem):
            pltpu.async_copy(qkt_hbm.at[rI.at[c]], qa, sem)
            pltpu.async_copy(qkt_hbm.at[cI.at[c]], kb, sem)
            pltpu.async_copy(v_hbm.at[cI.at[c]], vb, sem)

        def drain(c, qa, kb, vb, sem):
            pltpu.make_async_copy(qkt_hbm.at[rI.at[c]], qa, sem).wait()
            pltpu.make_async_copy(qkt_hbm.at[cI.at[c]], kb, sem).wait()
            pltpu.make_async_copy(v_hbm.at[cI.at[c]], vb, sem).wait()

        def compute(qa, kb, vb, wb):
            @plsc.parallel_loop(0, _CH, unroll=4)
            def _(i):
                a = qa[i, :]
                b = kb[i, :]
                e = jnp.exp(a * _dyn_gather(b, rot8))
                for k2 in range(4):
                    ev = _dyn_gather(e, hsel + 2 * k2)
                    wb[i, pl.ds(16 * k2, 16)] = vb[i, pl.ds(16 * k2, 16)] * ev
                wb[i, pl.ds(64, 16)] = jnp.where(lane < 8, e, 0.0)

        issue(0, qaA, kbA, vbA, sgA)
        issue(1, qaB, kbB, vbB, sgB)

        @pl.loop(0, _NCH // 2)
        def _(t):
            c0 = 2 * t
            c1 = 2 * t + 1
            drain(c0, qaA, kbA, vbA, sgA)
            compute(qaA, kbA, vbA, wbA)
            dsa = pltpu.async_copy(wbA, acc_sh.at[rI.at[c0]], ssA, add=True)

            @pl.when(t < _NCH // 2 - 1)
            def _():
                issue(c0 + 2, qaA, kbA, vbA, sgA)

            drain(c1, qaB, kbB, vbB, sgB)
            compute(qaB, kbB, vbB, wbB)
            dsb = pltpu.async_copy(wbB, acc_sh.at[rI.at[c1]], ssB, add=True)

            @pl.when(t < _NCH // 2 - 1)
            def _():
                issue(c1 + 2, qaB, kbB, vbB, sgB)

            dsa.wait()
            dsb.wait()

        plsc.subcore_barrier()
        pltpu.sync_copy(acc_sh.at[pl.ds(sid * 640, 640)],
                        out_hbm.at[pl.ds(cid * _NP + sid * 640, 640)])

    return k(rowp, colp, qkt, v0)


def _scl1(rowp, colp, q1t, k1t, v1):
    @functools.partial(
        pl.kernel,
        out_type=jax.ShapeDtypeStruct((2 * _NP, 64), _F32),
        mesh=_mesh(),
        compiler_params=_SC_PARAMS,
        scratch_types=[
            pltpu.VMEM_SHARED((_NP, 64), _F32),
            pltpu.VMEM((_NCH, _CH), jnp.int32),
            pltpu.VMEM((_NCH, _CH), jnp.int32),
            pltpu.VMEM((_NP,), _F32),
            pltpu.VMEM((_NP,), _F32),
            pltpu.VMEM((_CH, 48), _F32),
            pltpu.VMEM((_CH, 48), _F32),
            pltpu.VMEM((_CH, 64), _F32),
            pltpu.VMEM((_CH, 64), _F32),
            pltpu.SemaphoreType.DMA,
            pltpu.SemaphoreType.DMA,
            pltpu.SemaphoreType.DMA,
            pltpu.SemaphoreType.DMA,
            pltpu.SemaphoreType.DMA,
        ],
    )
    def k(row_hbm, col_hbm, q1_hbm, k1_hbm, v1_hbm, out_hbm,
          acc_sh, rI, cI, q1t_v, k1t_v, vbA, vbB, wbA, wbB,
          sgA, sgB, ssA, ssB, sidx):
        cid = lax.axis_index("c")
        sid = lax.axis_index("s")
        wid = cid * _NS + sid
        lane = lax.iota(jnp.int32, 16)
        zf = (lane * 0).astype(_F32)

        r0 = pl.multiple_of(wid * _NCH, 8)
        di1 = pltpu.async_copy(row_hbm.at[pl.ds(r0, _NCH)], rI, sidx)
        di2 = pltpu.async_copy(col_hbm.at[pl.ds(r0, _NCH)], cI, sidx)
        dq = pltpu.async_copy(q1_hbm, q1t_v, sidx)
        dk = pltpu.async_copy(k1_hbm, k1t_v, sidx)

        @pl.loop(0, _CH)
        def _(i):
            for k2 in range(4):
                wbA[i, pl.ds(16 * k2, 16)] = zf
        for m in range(5):
            pltpu.sync_copy(wbA, acc_sh.at[pl.ds(sid * 640 + m * _CH, _CH)])
        plsc.subcore_barrier()
        di1.wait()
        di2.wait()
        dq.wait()
        dk.wait()

        def issue(c, vb, sem):
            pltpu.async_copy(v1_hbm.at[cI.at[c]], vb, sem)

        def drain(c, vb, sem):
            pltpu.make_async_copy(v1_hbm.at[cI.at[c]], vb, sem).wait()

        def compute(c, vb, wb):
            @plsc.parallel_loop(0, _CH // 16, unroll=4)
            def _(g):
                idxr = rI[c, pl.ds(g * 16, 16)]
                idxc = cI[c, pl.ds(g * 16, 16)]
                qg = plsc.load_gather(q1t_v, [idxr])
                kg = plsc.load_gather(k1t_v, [idxc])
                e1 = jnp.exp(qg * kg)
                for i in range(16):
                    sp = _dyn_gather(e1, lane * 0 + i)
                    ei = g * 16 + i
                    for k2 in range(3):
                        wb[ei, pl.ds(16 * k2, 16)] = (
                            vb[ei, pl.ds(16 * k2, 16)] * sp)
                    wb[ei, pl.ds(48, 16)] = jnp.where(lane < 1, sp, 0.0)

        issue(0, vbA, sgA)
        issue(1, vbB, sgB)

        @pl.loop(0, _NCH // 2)
        def _(t):
            c0 = 2 * t
            c1 = 2 * t + 1
            drain(c0, vbA, sgA)
            compute(c0, vbA, wbA)
            dsa = pltpu.async_copy(wbA, acc_sh.at[rI.at[c0]], ssA, add=True)

            @pl.when(t < _NCH // 2 - 1)
            def _():
                issue(c0 + 2, vbA, sgA)

            drain(c1, vbB, sgB)
            compute(c1, vbB, wbB)
            dsb = pltpu.async_copy(wbB, acc_sh.at[rI.at[c1]], ssB, add=True)

            @pl.when(t < _NCH // 2 - 1)
            def _():
                issue(c1 + 2, vbB, sgB)

            dsa.wait()
            dsb.wait()

        plsc.subcore_barrier()
        pltpu.sync_copy(acc_sh.at[pl.ds(sid * 640, 640)],
                        out_hbm.at[pl.ds(cid * _NP + sid * 640, 640)])

    return k(rowp, colp, q1t, k1t, v1)


def kernel(x, edge_index, edge_weight, qk0, qb0, kk0, kb0, k0, b0,
           qk1, qb1, kk1, kb1, k1, b1):
    del edge_weight
    xp = jnp.zeros((_NP, 128), _F32).at[:_N].set(x)
    row = edge_index[0].astype(jnp.int32)
    col = edge_index[1].astype(jnp.int32)
    pad = _N + (jnp.arange(_EP - _E, dtype=jnp.int32) % (_NP - _N))
    rowp = jnp.concatenate([row, pad]).reshape(_EP // _CH, _CH)
    colp = jnp.concatenate([col, pad]).reshape(_EP // _CH, _CH)
    p8 = jnp.kron(jnp.eye(8, dtype=_F32), jnp.ones((1, 8), _F32))

    qkt, v0, es0, outs0 = _tc1(xp, qk0, qb0.reshape(1, 8), kk0,
                               kb0.reshape(1, 8), k0, p8)
    accp = _scl0(rowp, colp, qkt, v0)
    k1p = jnp.zeros((64, 48), _F32).at[:, :40].set(k1)
    q1, k1c, v1, es1, outs1 = _tc2(accp, es0, outs0, p8, b0.reshape(1, 64),
                                   qk1, qb1.reshape(1, 1), kk1,
                                   kb1.reshape(1, 1), k1p)
    acc1p = _scl1(rowp, colp, q1.reshape(_NP), k1c.reshape(_NP), v1)
    b1p = jnp.zeros((1, 48), _F32).at[0, :40].set(b1)
    out = _tc3(acc1p, es1, outs1, b1p)
    return out[:_N, :40]


# speedup vs baseline: 186.3563x; 1.0070x over previous
import functools

import jax
import jax.numpy as jnp
from jax import lax
from jax.experimental import pallas as pl
from jax.experimental.pallas import tpu as pltpu
from jax.experimental.pallas import tpu_sc as plsc

_N = 10000
_NP = 10240
_E = 320000
_NC, _NS = 2, 16
_NW = _NC * _NS
_EPW = 10240
_EP = _NW * _EPW
_CH = 128
_NCH = _EPW // _CH
_F32 = jnp.float32


def _dyn_gather(v, idx):
    dnums = lax.GatherDimensionNumbers(
        offset_dims=(), collapsed_slice_dims=(0,), start_index_map=(0,))
    return lax.gather(v, idx[:, None], dnums, (1,),
                      mode=lax.GatherScatterMode.PROMISE_IN_BOUNDS)


def _tc1(xp, qk0, qb0, kk0, kb0, k0, p8):
    def body(x_ref, qk_ref, qb_ref, kk_ref, kb_ref, kv_ref, p_ref,
             qkt_ref, v_ref, es_ref, outs_ref):
        x = x_ref[...]
        q = jnp.maximum(jnp.dot(x, qk_ref[...], preferred_element_type=_F32)
                        + qb_ref[...], 0.0)
        k = jnp.maximum(jnp.dot(x, kk_ref[...], preferred_element_type=_F32)
                        + kb_ref[...], 0.0)
        v = jnp.dot(x, kv_ref[...], preferred_element_type=_F32)
        es = jnp.exp(q * k)
        es64 = jnp.dot(es, p_ref[...], preferred_element_type=_F32)
        qkt_ref[...] = jnp.concatenate([q, k], axis=1)
        v_ref[...] = v
        es_ref[...] = es
        outs_ref[...] = es64 * v
    return pl.pallas_call(
        body,
        out_shape=(
            jax.ShapeDtypeStruct((_NP, 16), _F32),
            jax.ShapeDtypeStruct((_NP, 64), _F32),
            jax.ShapeDtypeStruct((_NP, 8), _F32),
            jax.ShapeDtypeStruct((_NP, 64), _F32),
        ),
    )(xp, qk0, qb0, kk0, kb0, k0, p8)


def _tc2(accp, es0, outs0, p8, b0, qk1, qb1, kk1, kb1, k1p):
    def body(a_ref, es_ref, os_ref, p_ref, b0_ref,
             qk_ref, qb_ref, kk_ref, kb_ref, kv_ref,
             q1_ref, k1_ref, v1_ref, es1_ref, outs1_ref):
        acc = a_ref[0:_NP, :] + a_ref[_NP:2 * _NP, :]
        s = acc[:, 64:72] + es_ref[...]
        s64 = jnp.dot(s, p_ref[...], preferred_element_type=_F32)
        h = jnp.maximum((acc[:, 0:64] + os_ref[...]) / s64 + b0_ref[...], 0.0)
        q1 = jnp.maximum(jnp.dot(h, qk_ref[...], preferred_element_type=_F32)
                         + qb_ref[...], 0.0)
        k1 = jnp.maximum(jnp.dot(h, kk_ref[...], preferred_element_type=_F32)
                         + kb_ref[...], 0.0)
        v1 = jnp.dot(h, kv_ref[...], preferred_element_type=_F32)
        es1 = jnp.exp(q1 * k1)
        q1_ref[...] = q1
        k1_ref[...] = k1
        v1_ref[...] = v1
        es1_ref[...] = es1
        outs1_ref[...] = es1 * v1
    return pl.pallas_call(
        body,
        out_shape=(
            jax.ShapeDtypeStruct((_NP, 1), _F32),
            jax.ShapeDtypeStruct((_NP, 1), _F32),
            jax.ShapeDtypeStruct((_NP, 48), _F32),
            jax.ShapeDtypeStruct((_NP, 1), _F32),
            jax.ShapeDtypeStruct((_NP, 48), _F32),
        ),
    )(accp, es0, outs0, p8, b0, qk1, qb1, kk1, kb1, k1p)


def _tc3(acc1p, es1, outs1, b1p):
    def body(a_ref, es_ref, os_ref, b_ref, o_ref):
        acc = a_ref[0:_NP, :] + a_ref[_NP:2 * _NP, :]
        s1 = acc[:, 48:49] + es_ref[...]
        o_ref[...] = (acc[:, 0:48] + os_ref[...]) / s1 + b_ref[...]
    return pl.pallas_call(
        body,
        out_shape=jax.ShapeDtypeStruct((_NP, 48), _F32),
    )(acc1p, es1, outs1, b1p)


def _mesh():
    return plsc.VectorSubcoreMesh(core_axis_name="c", subcore_axis_name="s",
                                  num_cores=_NC, num_subcores=_NS)


_SC_PARAMS = pltpu.CompilerParams(use_tc_tiling_on_sc=False,
                                  needs_layout_passes=False)


def _scl0(rowp, colp, qkt, v0):
    @functools.partial(
        pl.kernel,
        out_type=jax.ShapeDtypeStruct((2 * _NP, 80), _F32),
        mesh=_mesh(),
        compiler_params=_SC_PARAMS,
        scratch_types=[
            pltpu.VMEM_SHARED((_NP, 80), _F32),
            pltpu.VMEM((_NCH, _CH), jnp.int32),
            pltpu.VMEM((_NCH, _CH), jnp.int32),
            pltpu.VMEM((_CH, 16), _F32),
            pltpu.VMEM((_CH, 16), _F32),
            pltpu.VMEM((_CH, 16), _F32),
            pltpu.VMEM((_CH, 16), _F32),
            pltpu.VMEM((_CH, 64), _F32),
            pltpu.VMEM((_CH, 64), _F32),
            pltpu.VMEM((_CH, 80), _F32),
            pltpu.VMEM((_CH, 80), _F32),
            pltpu.SemaphoreType.DMA,
            pltpu.SemaphoreType.DMA,
            pltpu.SemaphoreType.DMA,
            pltpu.SemaphoreType.DMA,
            pltpu.SemaphoreType.DMA,
        ],
    )
    def k(row_hbm, col_hbm, qkt_hbm, v_hbm, out_hbm,
          acc_sh, rI, cI, qaA, qaB, kbA, kbB, vbA, vbB, wbA, wbB,
          sgA, sgB, ssA, ssB, sidx):
        cid = lax.axis_index("c")
        sid = lax.axis_index("s")
        wid = cid * _NS + sid
        lane = lax.iota(jnp.int32, 16)
        rot8 = lane ^ 8
        hsel = lane >> 3
        zf = (lane * 0).astype(_F32)

        r0 = pl.multiple_of(wid * _NCH, 8)
        di1 = pltpu.async_copy(row_hbm.at[pl.ds(r0, _NCH)], rI, sidx)
        di2 = pltpu.async_copy(col_hbm.at[pl.ds(r0, _NCH)], cI, sidx)

        @pl.loop(0, _CH)
        def _(i):
            for k2 in range(5):
                wbA[i, pl.ds(16 * k2, 16)] = zf
        for m in range(5):
            pltpu.sync_copy(wbA, acc_sh.at[pl.ds(sid * 640 + m * _CH, _CH)])
        plsc.subcore_barrier()
        di1.wait()
        di2.wait()

        def issue(c, qa, kb, vb, sem):
            pltpu.async_copy(qkt_hbm.at[rI.at[c]], qa, sem)
            pltpu.async_copy(qkt_hbm.at[cI.at[c]], kb, sem)
            pltpu.async_copy(v_hbm.at[cI.at[c]], vb, sem)

        def drain(c, qa, kb, vb, sem):
            pltpu.make_async_copy(qkt_hbm.at[rI.at[c]], qa, sem).wait()
            pltpu.make_async_copy(qkt_hbm.at[cI.at[c]], kb, sem).wait()
            pltpu.make_async_copy(v_hbm.at[cI.at[c]], vb, sem).wait()

        def compute(qa, kb, vb, wb):
            @plsc.parallel_loop(0, _CH, unroll=8)
            def _(i):
                a = qa[i, :]
                b = kb[i, :]
                e = jnp.exp(a * _dyn_gather(b, rot8))
                for k2 in range(4):
                    ev = _dyn_gather(e, hsel + 2 * k2)
                    wb[i, pl.ds(16 * k2, 16)] = vb[i, pl.ds(16 * k2, 16)] * ev
                wb[i, pl.ds(64, 16)] = jnp.where(lane < 8, e, 0.0)

        issue(0, qaA, kbA, vbA, sgA)
        issue(1, qaB, kbB, vbB, sgB)

        @pl.loop(0, _NCH // 2)
        def _(t):
            c0 = 2 * t
            c1 = 2 * t + 1
            drain(c0, qaA, kbA, vbA, sgA)
            compute(qaA, kbA, vbA, wbA)
            dsa = pltpu.async_copy(wbA, acc_sh.at[rI.at[c0]], ssA, add=True)

            @pl.when(t < _NCH // 2 - 1)
            def _():
                issue(c0 + 2, qaA, kbA, vbA, sgA)

            drain(c1, qaB, kbB, vbB, sgB)
            compute(qaB, kbB, vbB, wbB)
            dsb = pltpu.async_copy(wbB, acc_sh.at[rI.at[c1]], ssB, add=True)

            @pl.when(t < _NCH // 2 - 1)
            def _():
                issue(c1 + 2, qaB, kbB, vbB, sgB)

            dsa.wait()
            dsb.wait()

        plsc.subcore_barrier()
        pltpu.sync_copy(acc_sh.at[pl.ds(sid * 640, 640)],
                        out_hbm.at[pl.ds(cid * _NP + sid * 640, 640)])

    return k(rowp, colp, qkt, v0)


def _scl1(rowp, colp, q1t, k1t, v1):
    @functools.partial(
        pl.kernel,
        out_type=jax.ShapeDtypeStruct((2 * _NP, 64), _F32),
        mesh=_mesh(),
        compiler_params=_SC_PARAMS,
        scratch_types=[
            pltpu.VMEM_SHARED((_NP, 64), _F32),
            pltpu.VMEM((_NCH, _CH), jnp.int32),
            pltpu.VMEM((_NCH, _CH), jnp.int32),
            pltpu.VMEM((_NP,), _F32),
            pltpu.VMEM((_NP,), _F32),
            pltpu.VMEM((_CH, 48), _F32),
            pltpu.VMEM((_CH, 48), _F32),
            pltpu.VMEM((_CH, 64), _F32),
            pltpu.VMEM((_CH, 64), _F32),
            pltpu.SemaphoreType.DMA,
            pltpu.SemaphoreType.DMA,
            pltpu.SemaphoreType.DMA,
            pltpu.SemaphoreType.DMA,
            pltpu.SemaphoreType.DMA,
        ],
    )
    def k(row_hbm, col_hbm, q1_hbm, k1_hbm, v1_hbm, out_hbm,
          acc_sh, rI, cI, q1t_v, k1t_v, vbA, vbB, wbA, wbB,
          sgA, sgB, ssA, ssB, sidx):
        cid = lax.axis_index("c")
        sid = lax.axis_index("s")
        wid = cid * _NS + sid
        lane = lax.iota(jnp.int32, 16)
        zf = (lane * 0).astype(_F32)

        r0 = pl.multiple_of(wid * _NCH, 8)
        di1 = pltpu.async_copy(row_hbm.at[pl.ds(r0, _NCH)], rI, sidx)
        di2 = pltpu.async_copy(col_hbm.at[pl.ds(r0, _NCH)], cI, sidx)
        dq = pltpu.async_copy(q1_hbm, q1t_v, sidx)
        dk = pltpu.async_copy(k1_hbm, k1t_v, sidx)

        @pl.loop(0, _CH)
        def _(i):
            for k2 in range(4):
                wbA[i, pl.ds(16 * k2, 16)] = zf
        for m in range(5):
            pltpu.sync_copy(wbA, acc_sh.at[pl.ds(sid * 640 + m * _CH, _CH)])
        plsc.subcore_barrier()
        di1.wait()
        di2.wait()
        dq.wait()
        dk.wait()

        def issue(c, vb, sem):
            pltpu.async_copy(v1_hbm.at[cI.at[c]], vb, sem)

        def drain(c, vb, sem):
            pltpu.make_async_copy(v1_hbm.at[cI.at[c]], vb, sem).wait()

        def compute(c, vb, wb):
            @plsc.parallel_loop(0, _CH // 16, unroll=8)
            def _(g):
                idxr = rI[c, pl.ds(g * 16, 16)]
                idxc = cI[c, pl.ds(g * 16, 16)]
                qg = plsc.load_gather(q1t_v, [idxr])
                kg = plsc.load_gather(k1t_v, [idxc])
                e1 = jnp.exp(qg * kg)
                for i in range(16):
                    sp = _dyn_gather(e1, lane * 0 + i)
                    ei = g * 16 + i
                    for k2 in range(3):
                        wb[ei, pl.ds(16 * k2, 16)] = (
                            vb[ei, pl.ds(16 * k2, 16)] * sp)
                    wb[ei, pl.ds(48, 16)] = jnp.where(lane < 1, sp, 0.0)

        issue(0, vbA, sgA)
        issue(1, vbB, sgB)

        @pl.loop(0, _NCH // 2)
        def _(t):
            c0 = 2 * t
            c1 = 2 * t + 1
            drain(c0, vbA, sgA)
            compute(c0, vbA, wbA)
            dsa = pltpu.async_copy(wbA, acc_sh.at[rI.at[c0]], ssA, add=True)

            @pl.when(t < _NCH // 2 - 1)
            def _():
                issue(c0 + 2, vbA, sgA)

            drain(c1, vbB, sgB)
            compute(c1, vbB, wbB)
            dsb = pltpu.async_copy(wbB, acc_sh.at[rI.at[c1]], ssB, add=True)

            @pl.when(t < _NCH // 2 - 1)
            def _():
                issue(c1 + 2, vbB, sgB)

            dsa.wait()
            dsb.wait()

        plsc.subcore_barrier()
        pltpu.sync_copy(acc_sh.at[pl.ds(sid * 640, 640)],
                        out_hbm.at[pl.ds(cid * _NP + sid * 640, 640)])

    return k(rowp, colp, q1t, k1t, v1)


def kernel(x, edge_index, edge_weight, qk0, qb0, kk0, kb0, k0, b0,
           qk1, qb1, kk1, kb1, k1, b1):
    del edge_weight
    xp = jnp.zeros((_NP, 128), _F32).at[:_N].set(x)
    row = edge_index[0].astype(jnp.int32)
    col = edge_index[1].astype(jnp.int32)
    pad = _N + (jnp.arange(_EP - _E, dtype=jnp.int32) % (_NP - _N))
    rowp = jnp.concatenate([row, pad]).reshape(_EP // _CH, _CH)
    colp = jnp.concatenate([col, pad]).reshape(_EP // _CH, _CH)
    p8 = jnp.kron(jnp.eye(8, dtype=_F32), jnp.ones((1, 8), _F32))

    qkt, v0, es0, outs0 = _tc1(xp, qk0, qb0.reshape(1, 8), kk0,
                               kb0.reshape(1, 8), k0, p8)
    accp = _scl0(rowp, colp, qkt, v0)
    k1p = jnp.zeros((64, 48), _F32).at[:, :40].set(k1)
    q1, k1c, v1, es1, outs1 = _tc2(accp, es0, outs0, p8, b0.reshape(1, 64),
                                   qk1, qb1.reshape(1, 1), kk1,
                                   kb1.reshape(1, 1), k1p)
    acc1p = _scl1(rowp, colp, q1.reshape(_NP), k1c.reshape(_NP), v1)
    b1p = jnp.zeros((1, 48), _F32).at[0, :40].set(b1)
    out = _tc3(acc1p, es1, outs1, b1p)
    return out[:_N, :40]
